# baseline (speedup 1.0000x reference)
import jax
import jax.numpy as jnp
from jax.experimental import pallas as pl

N = 4096
E = 65536
N_Z = 20
HID = 64
NUM_LAYERS = 2
V = 1.0


def _combine_kernel(zg_ref, zl_ref, gamma_ref, out_ref):
    out_ref[...] = gamma_ref[0] * zg_ref[...] + zl_ref[...]


def _combine(z_g, z_l, gamma):
    return pl.pallas_call(
        _combine_kernel,
        out_shape=jax.ShapeDtypeStruct(z_g.shape, z_g.dtype),
    )(z_g, z_l, gamma)


def kernel(x, adj_values, distance, params, edge_index):
    rows = edge_index[0]
    cols = edge_index[1]

    def spmm(h):
        return jax.ops.segment_sum(adj_values[:, None] * h[cols], rows, num_segments=N)

    h = x
    for i in range(3):
        h = jax.nn.relu(h @ params['ae_enc_W%d' % i] + params['ae_enc_b%d' % i])
    z_ae = h @ params['ae_enc_W3'] + params['ae_enc_b3']

    h = jax.nn.relu(spmm(x @ params['gae_enc_W0']))
    h = jax.nn.relu(spmm(h @ params['gae_enc_W1']))
    z_egae = spmm(h @ params['gae_enc_W2'])
    z_egae_adj = jax.nn.sigmoid(z_egae @ z_egae.T)

    a = params['a']
    z_i = a * z_ae + (1.0 - a) * z_egae

    ef = distance[rows, cols][:, None]
    hv = jax.nn.relu(z_i @ params['gl_Wv'] + params['gl_bv'])
    he = jax.nn.relu(ef @ params['gl_We'] + params['gl_be'])
    for l in range(NUM_LAYERS):
        m = jnp.concatenate([hv[rows], hv[cols], he], axis=1)
        m = jax.nn.relu(m @ params['gl_Wm%d' % l] + params['gl_bm%d' % l])
        agg = jax.ops.segment_sum(m, cols, num_segments=N)
        hv = jax.nn.relu(hv + agg @ params['gl_Wu%d' % l] + params['gl_bu%d' % l])
    z_g = hv @ params['gl_Wout'] + params['gl_bout']

    z_l = spmm(z_i)
    z_tilde = _combine(z_g, z_l, params['gamma'])

    h = z_tilde
    for i in range(3):
        h = jax.nn.relu(h @ params['ae_dec_W%d' % i] + params['ae_dec_b%d' % i])
    x_hat = h @ params['ae_dec_W3'] + params['ae_dec_b3']

    h = jax.nn.relu(spmm(z_tilde @ params['gae_dec_W0']))
    h = jax.nn.relu(spmm(h @ params['gae_dec_W1']))
    z_hat = spmm(h @ params['gae_dec_W2'])
    z_hat_adj = jax.nn.sigmoid(z_hat @ z_hat.T)
    adj_hat = z_egae_adj + z_hat_adj

    cl = params['cluster']

    def soft(z):
        d = jnp.sum((z[:, None, :] - cl[None, :, :]) ** 2, axis=2)
        q = 1.0 / (1.0 + d / V)
        q = q ** ((V + 1.0) / 2.0)
        return q / jnp.sum(q, axis=1, keepdims=True)

    return (x_hat, z_hat, adj_hat, z_ae, z_egae, soft(z_tilde), soft(z_ae), soft(z_egae), z_tilde)


# speedup vs baseline: 1.0181x; 1.0181x over previous
import functools

import jax
import jax.numpy as jnp
from jax.experimental import pallas as pl

N = 4096
E = 65536
N_INPUT = 512
N_Z = 20
HID = 64
V = 1.0

RB = 256
EB = 1024


def _full(shape):
    return pl.BlockSpec(shape, lambda *_: (0,) * len(shape))


def _rows(bs, ncols):
    return pl.BlockSpec((bs, ncols), lambda i: (i, 0))


def _ae_enc_body(x_ref, w0, b0, w1, b1, w2, b2, w3, b3, wg0, zae_ref, g0_ref):
    h = x_ref[...]
    g0_ref[...] = jnp.dot(h, wg0[...], preferred_element_type=jnp.float32)
    h1 = jnp.maximum(jnp.dot(h, w0[...], preferred_element_type=jnp.float32) + b0[...], 0.0)
    h2 = jnp.maximum(jnp.dot(h1, w1[...], preferred_element_type=jnp.float32) + b1[...], 0.0)
    h3 = jnp.maximum(jnp.dot(h2, w2[...], preferred_element_type=jnp.float32) + b2[...], 0.0)
    zae_ref[...] = jnp.dot(h3, w3[...], preferred_element_type=jnp.float32) + b3[...]


def _ae_encode(x, p):
    ws = []
    specs = [_rows(RB, N_INPUT)]
    for i in range(4):
        w = p['ae_enc_W%d' % i]
        b = p['ae_enc_b%d' % i].reshape(1, -1)
        ws += [w, b]
        specs += [_full(w.shape), _full(b.shape)]
    ws.append(p['gae_enc_W0'])
    specs.append(_full(p['gae_enc_W0'].shape))
    return pl.pallas_call(
        _ae_enc_body,
        grid=(N // RB,),
        in_specs=specs,
        out_specs=[_rows(RB, N_Z), _rows(RB, 128)],
        out_shape=[jax.ShapeDtypeStruct((N, N_Z), jnp.float32),
                   jax.ShapeDtypeStruct((N, 128), jnp.float32)],
    )(x, *ws)


def _mm_body(h_ref, w_ref, o_ref, *, relu_in):
    h = h_ref[...]
    if relu_in:
        h = jnp.maximum(h, 0.0)
    o_ref[...] = jnp.dot(h, w_ref[...], preferred_element_type=jnp.float32)


def _relu_mm(h, w, relu_in=True):
    k = h.shape[1]
    return pl.pallas_call(
        functools.partial(_mm_body, relu_in=relu_in),
        grid=(N // RB,),
        in_specs=[_rows(RB, k), _full(w.shape)],
        out_specs=_rows(RB, w.shape[1]),
        out_shape=jax.ShapeDtypeStruct((N, w.shape[1]), jnp.float32),
    )(h, w)


def _comb_body(zae_ref, zegae_ref, a_ref, wv, bv, wra, wrb,
               zi_ref, hv_ref, hr_ref, hc_ref):
    a = a_ref[...]
    zi = a * zae_ref[...] + (1.0 - a) * zegae_ref[...]
    zi_ref[...] = zi
    hv = jnp.maximum(jnp.dot(zi, wv[...], preferred_element_type=jnp.float32) + bv[...], 0.0)
    hv_ref[...] = hv
    hr_ref[...] = jnp.dot(hv, wra[...], preferred_element_type=jnp.float32)
    hc_ref[...] = jnp.dot(hv, wrb[...], preferred_element_type=jnp.float32)


def _combine_hv(z_ae, z_egae, p):
    wm = p['gl_Wm0']
    wra, wrb = wm[:HID], wm[HID:2 * HID]
    args = (z_ae, z_egae, p['a'], p['gl_Wv'], p['gl_bv'].reshape(1, -1), wra, wrb)
    specs = [_rows(RB, N_Z), _rows(RB, N_Z), _rows(RB, N_Z),
             _full(args[3].shape), _full(args[4].shape), _full(wra.shape), _full(wrb.shape)]
    return pl.pallas_call(
        _comb_body,
        grid=(N // RB,),
        in_specs=specs,
        out_specs=[_rows(RB, N_Z)] + [_rows(RB, HID)] * 3,
        out_shape=[jax.ShapeDtypeStruct((N, N_Z), jnp.float32)]
                  + [jax.ShapeDtypeStruct((N, HID), jnp.float32)] * 3,
    )(*args)


def _he_body(ef_ref, we, be, wc0, bm0, wc1, bm1, he0_ref, he1_ref):
    he = jnp.maximum(ef_ref[...] * we[...] + be[...], 0.0)
    he0_ref[...] = jnp.dot(he, wc0[...], preferred_element_type=jnp.float32) + bm0[...]
    he1_ref[...] = jnp.dot(he, wc1[...], preferred_element_type=jnp.float32) + bm1[...]


def _he_tables(ef, p):
    args = (ef.reshape(E, 1), p['gl_We'], p['gl_be'].reshape(1, -1),
            p['gl_Wm0'][2 * HID:], p['gl_bm0'].reshape(1, -1),
            p['gl_Wm1'][2 * HID:], p['gl_bm1'].reshape(1, -1))
    specs = [_rows(EB, 1)] + [_full(a.shape) for a in args[1:]]
    return pl.pallas_call(
        _he_body,
        grid=(E // EB,),
        in_specs=specs,
        out_specs=[_rows(EB, HID)] * 2,
        out_shape=[jax.ShapeDtypeStruct((E, HID), jnp.float32)] * 2,
    )(*args)


def _upd_body(hv_ref, agg_ref, wu, bu, wra, wrb, hv1_ref, hr_ref, hc_ref):
    hv1 = jnp.maximum(
        hv_ref[...] + jnp.dot(agg_ref[...], wu[...], preferred_element_type=jnp.float32)
        + bu[...], 0.0)
    hv1_ref[...] = hv1
    hr_ref[...] = jnp.dot(hv1, wra[...], preferred_element_type=jnp.float32)
    hc_ref[...] = jnp.dot(hv1, wrb[...], preferred_element_type=jnp.float32)


def _hv_update(hv, agg, p):
    wm = p['gl_Wm1']
    wra, wrb = wm[:HID], wm[HID:2 * HID]
    args = (hv, agg, p['gl_Wu0'], p['gl_bu0'].reshape(1, -1), wra, wrb)
    specs = [_rows(RB, HID), _rows(RB, HID)] + [_full(a.shape) for a in args[2:]]
    return pl.pallas_call(
        _upd_body,
        grid=(N // RB,),
        in_specs=specs,
        out_specs=[_rows(RB, HID)] * 3,
        out_shape=[jax.ShapeDtypeStruct((N, HID), jnp.float32)] * 3,
    )(*args)


def _fin_body(hv_ref, agg_ref, zl_ref, wu, bu, wout, bout, gamma,
              w0, b0, w1, b1, w2, b2, w3, b3, wg0,
              zt_ref, xhat_ref, u0_ref):
    hv2 = jnp.maximum(
        hv_ref[...] + jnp.dot(agg_ref[...], wu[...], preferred_element_type=jnp.float32)
        + bu[...], 0.0)
    zg = jnp.dot(hv2, wout[...], preferred_element_type=jnp.float32) + bout[...]
    zt = gamma[...] * zg + zl_ref[...]
    zt_ref[...] = zt
    u0_ref[...] = jnp.dot(zt, wg0[...], preferred_element_type=jnp.float32)
    h1 = jnp.maximum(jnp.dot(zt, w0[...], preferred_element_type=jnp.float32) + b0[...], 0.0)
    h2 = jnp.maximum(jnp.dot(h1, w1[...], preferred_element_type=jnp.float32) + b1[...], 0.0)
    h3 = jnp.maximum(jnp.dot(h2, w2[...], preferred_element_type=jnp.float32) + b2[...], 0.0)
    xhat_ref[...] = jnp.dot(h3, w3[...], preferred_element_type=jnp.float32) + b3[...]


def _final_update(hv1, agg1, z_l, p):
    args = [hv1, agg1, z_l, p['gl_Wu1'], p['gl_bu1'].reshape(1, -1),
            p['gl_Wout'], p['gl_bout'].reshape(1, -1), p['gamma'].reshape(1, 1)]
    for i in range(4):
        args += [p['ae_dec_W%d' % i], p['ae_dec_b%d' % i].reshape(1, -1)]
    args.append(p['gae_dec_W0'])
    specs = [_rows(RB, HID), _rows(RB, HID), _rows(RB, N_Z)] \
        + [_full(a.shape) for a in args[3:]]
    return pl.pallas_call(
        _fin_body,
        grid=(N // RB,),
        in_specs=specs,
        out_specs=[_rows(RB, N_Z), _rows(RB, N_INPUT), _rows(RB, 256)],
        out_shape=[jax.ShapeDtypeStruct((N, N_Z), jnp.float32),
                   jax.ShapeDtypeStruct((N, N_INPUT), jnp.float32),
                   jax.ShapeDtypeStruct((N, 256), jnp.float32)],
    )(*args)


AB = 512


def _adj_body(zi_ref, zj_ref, hi_ref, hj_ref, o_ref):
    s1 = jax.lax.dot_general(zi_ref[...], zj_ref[...],
                             (((1,), (1,)), ((), ())),
                             preferred_element_type=jnp.float32)
    s2 = jax.lax.dot_general(hi_ref[...], hj_ref[...],
                             (((1,), (1,)), ((), ())),
                             preferred_element_type=jnp.float32)
    o_ref[...] = jax.nn.sigmoid(s1) + jax.nn.sigmoid(s2)


def _adj_hat(z_egae, z_hat):
    return pl.pallas_call(
        _adj_body,
        grid=(N // AB, N // AB),
        in_specs=[pl.BlockSpec((AB, N_Z), lambda i, j: (i, 0)),
                  pl.BlockSpec((AB, N_Z), lambda i, j: (j, 0)),
                  pl.BlockSpec((AB, N_INPUT), lambda i, j: (i, 0)),
                  pl.BlockSpec((AB, N_INPUT), lambda i, j: (j, 0))],
        out_specs=pl.BlockSpec((AB, AB), lambda i, j: (i, j)),
        out_shape=jax.ShapeDtypeStruct((N, N), jnp.float32),
    )(z_egae, z_egae, z_hat, z_hat)


def _soft_body(z1_ref, z2_ref, z3_ref, cl_ref, cn_ref, q1_ref, q2_ref, q3_ref):
    cl = cl_ref[...]
    cn = cn_ref[...]
    for z_ref, q_ref in ((z1_ref, q1_ref), (z2_ref, q2_ref), (z3_ref, q3_ref)):
        z = z_ref[...]
        zn = jnp.sum(z * z, axis=1, keepdims=True)
        d = zn + cn - 2.0 * jax.lax.dot_general(
            z, cl, (((1,), (1,)), ((), ())), preferred_element_type=jnp.float32)
        q = 1.0 / (1.0 + d / V)
        q_ref[...] = q / jnp.sum(q, axis=1, keepdims=True)


def _softs(z_tilde, z_ae, z_egae, cl):
    cn = jnp.sum(cl * cl, axis=1).reshape(1, -1)
    nc = cl.shape[0]
    return pl.pallas_call(
        _soft_body,
        grid=(N // AB,),
        in_specs=[_rows(AB, N_Z)] * 3 + [_full(cl.shape), _full(cn.shape)],
        out_specs=[_rows(AB, nc)] * 3,
        out_shape=[jax.ShapeDtypeStruct((N, nc), jnp.float32)] * 3,
    )(z_tilde, z_ae, z_egae, cl, cn)


def kernel(x, adj_values, distance, params, edge_index):
    p = params
    rows = edge_index[0]
    cols = edge_index[1]

    def spmm(h):
        return jax.ops.segment_sum(adj_values[:, None] * h[cols], rows, num_segments=N)

    z_ae, g0 = _ae_encode(x, p)

    g1 = _relu_mm(spmm(g0), p['gae_enc_W1'])
    g2 = _relu_mm(spmm(g1), p['gae_enc_W2'])
    z_egae = spmm(g2)

    z_i, hv, hr0, hc0 = _combine_hv(z_ae, z_egae, p)
    z_l = spmm(z_i)

    ef = distance[rows, cols]
    he0, he1 = _he_tables(ef, p)

    m0 = jnp.maximum(hr0[rows] + hc0[cols] + he0, 0.0)
    agg0 = jax.ops.segment_sum(m0, cols, num_segments=N)
    hv1, hr1, hc1 = _hv_update(hv, agg0, p)

    m1 = jnp.maximum(hr1[rows] + hc1[cols] + he1, 0.0)
    agg1 = jax.ops.segment_sum(m1, cols, num_segments=N)

    z_tilde, x_hat, u0 = _final_update(hv1, agg1, z_l, p)

    u1 = _relu_mm(spmm(u0), p['gae_dec_W1'])
    u2 = _relu_mm(spmm(u1), p['gae_dec_W2'])
    z_hat = spmm(u2)

    adj_hat = _adj_hat(z_egae, z_hat)
    q1, q2, q3 = _softs(z_tilde, z_ae, z_egae, p['cluster'])

    return (x_hat, z_hat, adj_hat, z_ae, z_egae, q1, q2, q3, z_tilde)


# speedup vs baseline: 2.5127x; 2.4680x over previous
import functools

import jax
import jax.numpy as jnp
from jax import lax
from jax.experimental import pallas as pl
from jax.experimental.pallas import tpu as pltpu
from jax.experimental.pallas import tpu_sc as plsc

N = 4096
E = 65536
N_INPUT = 512
N_Z = 20
HID = 64
V = 1.0

RB = 256
EB = 1024


def _full(shape):
    return pl.BlockSpec(shape, lambda *_: (0,) * len(shape))


def _rows(bs, ncols):
    return pl.BlockSpec((bs, ncols), lambda i: (i, 0))


GR = 256
NG = N // GR
ACC_LEN = (GR + 2) * N
DUMP = GR * N
EPT = E // 16
EFT = E // 32
ZB = 16384
TPW = ACC_LEN // 16


def _abuild_body(rows_hbm, cols_hbm, vals_hbm, dist_hbm, a_hbm, ef_hbm,
                 rows_v, cols_v, vals_v, idx_v, ef_v, zbuf, acc):
    c = lax.axis_index("c")
    s = lax.axis_index("s")
    ebase = s * EPT
    pltpu.sync_copy(rows_hbm.at[pl.ds(ebase, EPT)], rows_v)
    pltpu.sync_copy(cols_hbm.at[pl.ds(ebase, EPT)], cols_v)
    pltpu.sync_copy(vals_hbm.at[pl.ds(ebase, EPT)], vals_v)

    def zb_init(i, _):
        zbuf[pl.ds(i * 16, 16)] = jnp.zeros((16,), jnp.float32)
        return 0
    lax.fori_loop(0, ZB // 16, zb_init, 0)

    off0 = c * EFT

    def ef_chunk(j, _):
        def lanes(l, _):
            o = off0 + j * 128 + l * 16
            r = rows_v[pl.ds(o, 16)]
            cc = cols_v[pl.ds(o, 16)]
            row = idx_v.at[0]
            row[pl.ds(l * 16, 16)] = r * N + cc
            return 0
        lax.fori_loop(0, 8, lanes, 0)
        pltpu.sync_copy(dist_hbm.at[idx_v.at[0]], ef_v)
        pltpu.sync_copy(ef_v, ef_hbm.at[pl.ds(ebase + off0 + j * 128, 128)])
        return 0
    lax.fori_loop(0, EFT // 128, ef_chunk, 0)

    def group(g, _):
        lo = (g * 2 + c) * GR
        base = s * TPW
        for t in range(4):
            pltpu.sync_copy(zbuf, acc.at[pl.ds(base + t * ZB, ZB)])
        pltpu.sync_copy(zbuf.at[pl.ds(0, TPW - 4 * ZB)],
                        acc.at[pl.ds(base + 4 * ZB, TPW - 4 * ZB)])
        plsc.subcore_barrier()

        def chunk(j, _):
            def lanes(l, _):
                o = j * 128 + l * 16
                r = rows_v[pl.ds(o, 16)]
                cc = cols_v[pl.ds(o, 16)]
                rel = r - lo
                ing = (rel >= 0) & (rel < GR)
                row = idx_v.at[1]
                row[pl.ds(l * 16, 16)] = jnp.where(ing, rel * N + cc, DUMP + cc)
                return 0
            lax.fori_loop(0, 8, lanes, 0)
            pltpu.sync_copy(vals_v.at[pl.ds(j * 128, 128)],
                            acc.at[idx_v.at[1]], add=True)
            return 0
        lax.fori_loop(0, EPT // 128, chunk, 0)
        plsc.subcore_barrier()
        pltpu.sync_copy(acc.at[pl.ds(s * (16 * N), 16 * N)],
                        a_hbm.at[pl.ds((lo + s * 16) * N, 16 * N)])
        plsc.subcore_barrier()
        return 0
    lax.fori_loop(0, NG // 2, group, 0)


def _sc_build(rows, cols, vals, dist_flat):
    mesh = plsc.VectorSubcoreMesh(core_axis_name="c", subcore_axis_name="s",
                                  num_cores=2, num_subcores=16)
    f = pl.kernel(
        _abuild_body,
        out_type=[jax.ShapeDtypeStruct((N * N,), jnp.float32),
                  jax.ShapeDtypeStruct((E,), jnp.float32)],
        mesh=mesh,
        scratch_types=[
            pltpu.VMEM((EPT,), jnp.int32),
            pltpu.VMEM((EPT,), jnp.int32),
            pltpu.VMEM((EPT,), jnp.float32),
            pltpu.VMEM((2, 128), jnp.int32),
            pltpu.VMEM((128,), jnp.float32),
            pltpu.VMEM((ZB,), jnp.float32),
            pltpu.VMEM_SHARED((ACC_LEN,), jnp.float32),
        ],
    )
    return f(rows, cols, vals, dist_flat)


def _spmm_body(a_ref, h_ref, w_ref, o_ref, *, relu):
    s = jnp.dot(a_ref[...], h_ref[...], preferred_element_type=jnp.float32)
    if relu:
        s = jnp.maximum(s, 0.0)
    if w_ref is not None:
        s = jnp.dot(s, w_ref[...], preferred_element_type=jnp.float32)
    o_ref[...] = s


def _spmm_mm(a, h, w=None, relu=False):
    d = h.shape[1]
    dout = d if w is None else w.shape[1]
    if w is None:
        body = functools.partial(lambda ar, hr, orr, relu: _spmm_body(ar, hr, None, orr, relu=relu), relu=relu)
        in_specs = [_rows(RB, N), _full((N, d))]
        args = (a, h)
    else:
        body = functools.partial(_spmm_body, relu=relu)
        in_specs = [_rows(RB, N), _full((N, d)), _full(w.shape)]
        args = (a, h, w)
    return pl.pallas_call(
        body,
        grid=(N // RB,),
        in_specs=in_specs,
        out_specs=_rows(RB, dout),
        out_shape=jax.ShapeDtypeStruct((N, dout), jnp.float32),
    )(*args)


def _ae_enc_body(x_ref, w0, b0, w1, b1, w2, b2, w3, b3, wg0, zae_ref, g0_ref):
    h = x_ref[...]
    g0_ref[...] = jnp.dot(h, wg0[...], preferred_element_type=jnp.float32)
    h1 = jnp.maximum(jnp.dot(h, w0[...], preferred_element_type=jnp.float32) + b0[...], 0.0)
    h2 = jnp.maximum(jnp.dot(h1, w1[...], preferred_element_type=jnp.float32) + b1[...], 0.0)
    h3 = jnp.maximum(jnp.dot(h2, w2[...], preferred_element_type=jnp.float32) + b2[...], 0.0)
    zae_ref[...] = jnp.dot(h3, w3[...], preferred_element_type=jnp.float32) + b3[...]


def _ae_encode(x, p):
    ws = []
    specs = [_rows(RB, N_INPUT)]
    for i in range(4):
        w = p['ae_enc_W%d' % i]
        b = p['ae_enc_b%d' % i].reshape(1, -1)
        ws += [w, b]
        specs += [_full(w.shape), _full(b.shape)]
    ws.append(p['gae_enc_W0'])
    specs.append(_full(p['gae_enc_W0'].shape))
    return pl.pallas_call(
        _ae_enc_body,
        grid=(N // RB,),
        in_specs=specs,
        out_specs=[_rows(RB, N_Z), _rows(RB, 128)],
        out_shape=[jax.ShapeDtypeStruct((N, N_Z), jnp.float32),
                   jax.ShapeDtypeStruct((N, 128), jnp.float32)],
    )(x, *ws)


def _mm_body(h_ref, w_ref, o_ref, *, relu_in):
    h = h_ref[...]
    if relu_in:
        h = jnp.maximum(h, 0.0)
    o_ref[...] = jnp.dot(h, w_ref[...], preferred_element_type=jnp.float32)


def _relu_mm(h, w, relu_in=True):
    k = h.shape[1]
    return pl.pallas_call(
        functools.partial(_mm_body, relu_in=relu_in),
        grid=(N // RB,),
        in_specs=[_rows(RB, k), _full(w.shape)],
        out_specs=_rows(RB, w.shape[1]),
        out_shape=jax.ShapeDtypeStruct((N, w.shape[1]), jnp.float32),
    )(h, w)


def _comb_body(zae_ref, zegae_ref, a_ref, wv, bv, wra, wrb,
               zi_ref, hv_ref, hr_ref, hc_ref):
    a = a_ref[...]
    zi = a * zae_ref[...] + (1.0 - a) * zegae_ref[...]
    zi_ref[...] = zi
    hv = jnp.maximum(jnp.dot(zi, wv[...], preferred_element_type=jnp.float32) + bv[...], 0.0)
    hv_ref[...] = hv
    hr_ref[...] = jnp.dot(hv, wra[...], preferred_element_type=jnp.float32)
    hc_ref[...] = jnp.dot(hv, wrb[...], preferred_element_type=jnp.float32)


def _combine_hv(z_ae, z_egae, p):
    wm = p['gl_Wm0']
    wra, wrb = wm[:HID], wm[HID:2 * HID]
    args = (z_ae, z_egae, p['a'], p['gl_Wv'], p['gl_bv'].reshape(1, -1), wra, wrb)
    specs = [_rows(RB, N_Z), _rows(RB, N_Z), _rows(RB, N_Z),
             _full(args[3].shape), _full(args[4].shape), _full(wra.shape), _full(wrb.shape)]
    return pl.pallas_call(
        _comb_body,
        grid=(N // RB,),
        in_specs=specs,
        out_specs=[_rows(RB, N_Z)] + [_rows(RB, HID)] * 3,
        out_shape=[jax.ShapeDtypeStruct((N, N_Z), jnp.float32)]
                  + [jax.ShapeDtypeStruct((N, HID), jnp.float32)] * 3,
    )(*args)


def _he_body(ef_ref, we, be, wc0, bm0, wc1, bm1, he0_ref, he1_ref):
    he = jnp.maximum(ef_ref[...] * we[...] + be[...], 0.0)
    he0_ref[...] = jnp.dot(he, wc0[...], preferred_element_type=jnp.float32) + bm0[...]
    he1_ref[...] = jnp.dot(he, wc1[...], preferred_element_type=jnp.float32) + bm1[...]


def _he_tables(ef, p):
    args = (ef.reshape(E, 1), p['gl_We'], p['gl_be'].reshape(1, -1),
            p['gl_Wm0'][2 * HID:], p['gl_bm0'].reshape(1, -1),
            p['gl_Wm1'][2 * HID:], p['gl_bm1'].reshape(1, -1))
    specs = [_rows(EB, 1)] + [_full(a.shape) for a in args[1:]]
    return pl.pallas_call(
        _he_body,
        grid=(E // EB,),
        in_specs=specs,
        out_specs=[_rows(EB, HID)] * 2,
        out_shape=[jax.ShapeDtypeStruct((E, HID), jnp.float32)] * 2,
    )(*args)


def _upd_body(hv_ref, agg_ref, wu, bu, wra, wrb, hv1_ref, hr_ref, hc_ref):
    hv1 = jnp.maximum(
        hv_ref[...] + jnp.dot(agg_ref[...], wu[...], preferred_element_type=jnp.float32)
        + bu[...], 0.0)
    hv1_ref[...] = hv1
    hr_ref[...] = jnp.dot(hv1, wra[...], preferred_element_type=jnp.float32)
    hc_ref[...] = jnp.dot(hv1, wrb[...], preferred_element_type=jnp.float32)


def _hv_update(hv, agg, p):
    wm = p['gl_Wm1']
    wra, wrb = wm[:HID], wm[HID:2 * HID]
    args = (hv, agg, p['gl_Wu0'], p['gl_bu0'].reshape(1, -1), wra, wrb)
    specs = [_rows(RB, HID), _rows(RB, HID)] + [_full(a.shape) for a in args[2:]]
    return pl.pallas_call(
        _upd_body,
        grid=(N // RB,),
        in_specs=specs,
        out_specs=[_rows(RB, HID)] * 3,
        out_shape=[jax.ShapeDtypeStruct((N, HID), jnp.float32)] * 3,
    )(*args)


def _fin_body(hv_ref, agg_ref, zl_ref, wu, bu, wout, bout, gamma,
              w0, b0, w1, b1, w2, b2, w3, b3, wg0,
              zt_ref, xhat_ref, u0_ref):
    hv2 = jnp.maximum(
        hv_ref[...] + jnp.dot(agg_ref[...], wu[...], preferred_element_type=jnp.float32)
        + bu[...], 0.0)
    zg = jnp.dot(hv2, wout[...], preferred_element_type=jnp.float32) + bout[...]
    zt = gamma[...] * zg + zl_ref[...]
    zt_ref[...] = zt
    u0_ref[...] = jnp.dot(zt, wg0[...], preferred_element_type=jnp.float32)
    h1 = jnp.maximum(jnp.dot(zt, w0[...], preferred_element_type=jnp.float32) + b0[...], 0.0)
    h2 = jnp.maximum(jnp.dot(h1, w1[...], preferred_element_type=jnp.float32) + b1[...], 0.0)
    h3 = jnp.maximum(jnp.dot(h2, w2[...], preferred_element_type=jnp.float32) + b2[...], 0.0)
    xhat_ref[...] = jnp.dot(h3, w3[...], preferred_element_type=jnp.float32) + b3[...]


def _final_update(hv1, agg1, z_l, p):
    args = [hv1, agg1, z_l, p['gl_Wu1'], p['gl_bu1'].reshape(1, -1),
            p['gl_Wout'], p['gl_bout'].reshape(1, -1), p['gamma'].reshape(1, 1)]
    for i in range(4):
        args += [p['ae_dec_W%d' % i], p['ae_dec_b%d' % i].reshape(1, -1)]
    args.append(p['gae_dec_W0'])
    specs = [_rows(RB, HID), _rows(RB, HID), _rows(RB, N_Z)] \
        + [_full(a.shape) for a in args[3:]]
    return pl.pallas_call(
        _fin_body,
        grid=(N // RB,),
        in_specs=specs,
        out_specs=[_rows(RB, N_Z), _rows(RB, N_INPUT), _rows(RB, 256)],
        out_shape=[jax.ShapeDtypeStruct((N, N_Z), jnp.float32),
                   jax.ShapeDtypeStruct((N, N_INPUT), jnp.float32),
                   jax.ShapeDtypeStruct((N, 256), jnp.float32)],
    )(*args)


AB = 512


def _adj_body(zi_ref, zj_ref, hi_ref, hj_ref, o_ref):
    s1 = jax.lax.dot_general(zi_ref[...], zj_ref[...],
                             (((1,), (1,)), ((), ())),
                             preferred_element_type=jnp.float32)
    s2 = jax.lax.dot_general(hi_ref[...], hj_ref[...],
                             (((1,), (1,)), ((), ())),
                             preferred_element_type=jnp.float32)
    o_ref[...] = jax.nn.sigmoid(s1) + jax.nn.sigmoid(s2)


def _adj_hat(z_egae, z_hat):
    return pl.pallas_call(
        _adj_body,
        grid=(N // AB, N // AB),
        in_specs=[pl.BlockSpec((AB, N_Z), lambda i, j: (i, 0)),
                  pl.BlockSpec((AB, N_Z), lambda i, j: (j, 0)),
                  pl.BlockSpec((AB, N_INPUT), lambda i, j: (i, 0)),
                  pl.BlockSpec((AB, N_INPUT), lambda i, j: (j, 0))],
        out_specs=pl.BlockSpec((AB, AB), lambda i, j: (i, j)),
        out_shape=jax.ShapeDtypeStruct((N, N), jnp.float32),
    )(z_egae, z_egae, z_hat, z_hat)


def _soft_body(z1_ref, z2_ref, z3_ref, cl_ref, cn_ref, q1_ref, q2_ref, q3_ref):
    cl = cl_ref[...]
    cn = cn_ref[...]
    for z_ref, q_ref in ((z1_ref, q1_ref), (z2_ref, q2_ref), (z3_ref, q3_ref)):
        z = z_ref[...]
        zn = jnp.sum(z * z, axis=1, keepdims=True)
        d = zn + cn - 2.0 * jax.lax.dot_general(
            z, cl, (((1,), (1,)), ((), ())), preferred_element_type=jnp.float32)
        q = 1.0 / (1.0 + d / V)
        q_ref[...] = q / jnp.sum(q, axis=1, keepdims=True)


def _softs(z_tilde, z_ae, z_egae, cl):
    cn = jnp.sum(cl * cl, axis=1).reshape(1, -1)
    nc = cl.shape[0]
    return pl.pallas_call(
        _soft_body,
        grid=(N // AB,),
        in_specs=[_rows(AB, N_Z)] * 3 + [_full(cl.shape), _full(cn.shape)],
        out_specs=[_rows(AB, nc)] * 3,
        out_shape=[jax.ShapeDtypeStruct((N, nc), jnp.float32)] * 3,
    )(z_tilde, z_ae, z_egae, cl, cn)


def kernel(x, adj_values, distance, params, edge_index):
    p = params
    rows = edge_index[0].astype(jnp.int32)
    cols = edge_index[1].astype(jnp.int32)

    a_flat, ef = _sc_build(rows, cols, adj_values, distance.reshape(-1))
    A = a_flat.reshape(N, N)

    z_ae, g0 = _ae_encode(x, p)

    g1 = _spmm_mm(A, g0, p['gae_enc_W1'], relu=True)
    g2 = _spmm_mm(A, g1, p['gae_enc_W2'], relu=True)
    z_egae = _spmm_mm(A, g2)

    z_i, hv, hr0, hc0 = _combine_hv(z_ae, z_egae, p)
    z_l = _spmm_mm(A, z_i)

    he0, he1 = _he_tables(ef, p)

    m0 = jnp.maximum(hr0[rows] + hc0[cols] + he0, 0.0)
    agg0 = jax.ops.segment_sum(m0, cols, num_segments=N)
    hv1, hr1, hc1 = _hv_update(hv, agg0, p)

    m1 = jnp.maximum(hr1[rows] + hc1[cols] + he1, 0.0)
    agg1 = jax.ops.segment_sum(m1, cols, num_segments=N)

    z_tilde, x_hat, u0 = _final_update(hv1, agg1, z_l, p)

    u1 = _spmm_mm(A, u0, p['gae_dec_W1'], relu=True)
    u2 = _spmm_mm(A, u1, p['gae_dec_W2'], relu=True)
    z_hat = _spmm_mm(A, u2)

    adj_hat = _adj_hat(z_egae, z_hat)
    q1, q2, q3 = _softs(z_tilde, z_ae, z_egae, p['cluster'])

    return (x_hat, z_hat, adj_hat, z_ae, z_egae, q1, q2, q3, z_tilde)


# speedup vs baseline: 5.3793x; 2.1408x over previous
import functools

import jax
import jax.numpy as jnp
from jax import lax
from jax.experimental import pallas as pl
from jax.experimental.pallas import tpu as pltpu
from jax.experimental.pallas import tpu_sc as plsc

N = 4096
E = 65536
N_INPUT = 512
N_Z = 20
HID = 64
V = 1.0

RB = 256
EB = 1024


def _full(shape):
    return pl.BlockSpec(shape, lambda *_: (0,) * len(shape))


def _rows(bs, ncols):
    return pl.BlockSpec((bs, ncols), lambda i: (i, 0))


GR = 256
NG = N // GR
ACC_LEN = (GR + 2) * N
DUMP = GR * N
EPT = E // 16
EFT = E // 32
ZB = 16384
TPW = ACC_LEN // 16


def _abuild_body(rows_hbm, cols_hbm, vals_hbm, dist_hbm, a_hbm, ef_hbm,
                 rows_v, cols_v, vals_v, idx_v, ef_v, zbuf, acc):
    c = lax.axis_index("c")
    s = lax.axis_index("s")
    ebase = s * EPT
    pltpu.sync_copy(rows_hbm.at[pl.ds(ebase, EPT)], rows_v)
    pltpu.sync_copy(cols_hbm.at[pl.ds(ebase, EPT)], cols_v)
    pltpu.sync_copy(vals_hbm.at[pl.ds(ebase, EPT)], vals_v)

    def zb_init(i, _):
        zbuf[pl.ds(i * 16, 16)] = jnp.zeros((16,), jnp.float32)
        return 0
    lax.fori_loop(0, ZB // 16, zb_init, 0)

    off0 = c * EFT

    def ef_chunk(j, _):
        def lanes(l, _):
            o = off0 + j * 128 + l * 16
            r = rows_v[pl.ds(o, 16)]
            cc = cols_v[pl.ds(o, 16)]
            row = idx_v.at[0]
            row[pl.ds(l * 16, 16)] = r * N + cc
            return 0
        lax.fori_loop(0, 8, lanes, 0)
        pltpu.sync_copy(dist_hbm.at[idx_v.at[0]], ef_v)
        pltpu.sync_copy(ef_v, ef_hbm.at[pl.ds(ebase + off0 + j * 128, 128)])
        return 0
    lax.fori_loop(0, EFT // 128, ef_chunk, 0)

    def group(g, _):
        lo = (g * 2 + c) * GR
        base = s * TPW
        for t in range(4):
            pltpu.sync_copy(zbuf, acc.at[pl.ds(base + t * ZB, ZB)])
        pltpu.sync_copy(zbuf.at[pl.ds(0, TPW - 4 * ZB)],
                        acc.at[pl.ds(base + 4 * ZB, TPW - 4 * ZB)])
        plsc.subcore_barrier()

        def chunk(j, _):
            def lanes(l, _):
                o = j * 128 + l * 16
                r = rows_v[pl.ds(o, 16)]
                cc = cols_v[pl.ds(o, 16)]
                rel = r - lo
                ing = (rel >= 0) & (rel < GR)
                row = idx_v.at[1]
                row[pl.ds(l * 16, 16)] = jnp.where(ing, rel * N + cc, DUMP + cc)
                return 0
            lax.fori_loop(0, 8, lanes, 0)
            pltpu.sync_copy(vals_v.at[pl.ds(j * 128, 128)],
                            acc.at[idx_v.at[1]], add=True)
            return 0
        lax.fori_loop(0, EPT // 128, chunk, 0)
        plsc.subcore_barrier()
        pltpu.sync_copy(acc.at[pl.ds(s * (16 * N), 16 * N)],
                        a_hbm.at[pl.ds((lo + s * 16) * N, 16 * N)])
        plsc.subcore_barrier()
        return 0
    lax.fori_loop(0, NG // 2, group, 0)


def _sc_build(rows, cols, vals, dist_flat):
    mesh = plsc.VectorSubcoreMesh(core_axis_name="c", subcore_axis_name="s",
                                  num_cores=2, num_subcores=16)
    f = pl.kernel(
        _abuild_body,
        out_type=[jax.ShapeDtypeStruct((N * N,), jnp.float32),
                  jax.ShapeDtypeStruct((E,), jnp.float32)],
        mesh=mesh,
        scratch_types=[
            pltpu.VMEM((EPT,), jnp.int32),
            pltpu.VMEM((EPT,), jnp.int32),
            pltpu.VMEM((EPT,), jnp.float32),
            pltpu.VMEM((2, 128), jnp.int32),
            pltpu.VMEM((128,), jnp.float32),
            pltpu.VMEM((ZB,), jnp.float32),
            pltpu.VMEM_SHARED((ACC_LEN,), jnp.float32),
        ],
    )
    return f(rows, cols, vals, dist_flat)


EMT = E // 32
ECH = EMT // 128


HIDP = 128


def _edge_mp_body(rows_hbm, cols_hbm, hr_hbm, hc_hbm, he_hbm, agg_hbm,
                  ridx, cidx, ga, gb, mb, zbuf, acc):
    c = lax.axis_index("c")
    s = lax.axis_index("s")
    tid = s * 2 + c
    ebase = tid * EMT

    def ld(j, _):
        pltpu.sync_copy(rows_hbm.at[pl.ds(ebase + j * 128, 128)], ridx.at[j])
        pltpu.sync_copy(cols_hbm.at[pl.ds(ebase + j * 128, 128)], cidx.at[j])
        return 0
    lax.fori_loop(0, ECH, ld, 0)

    def zb_init(i, _):
        def inner(k, _):
            zbuf[i, pl.ds(k * 16, 16)] = jnp.zeros((16,), jnp.float32)
            return 0
        lax.fori_loop(0, HIDP // 16, inner, 0)
        return 0
    lax.fori_loop(0, 256, zb_init, 0)
    pltpu.sync_copy(zbuf, acc.at[pl.ds(s * 256, 256)])
    plsc.subcore_barrier()

    def chunk(j, _):
        pltpu.sync_copy(hr_hbm.at[ridx.at[j]], ga)
        pltpu.sync_copy(hc_hbm.at[cidx.at[j]], gb)
        pltpu.sync_copy(he_hbm.at[pl.ds(ebase + j * 128, 128)], mb)

        def row(i, _):
            for k in range(HID // 16):
                d = pl.ds(k * 16, 16)
                mb[i, d] = jnp.maximum(ga[i, d] + gb[i, d] + mb[i, d], 0.0)
            return 0
        lax.fori_loop(0, 128, row, 0)
        pltpu.sync_copy(mb, acc.at[cidx.at[j]], add=True)
        return 0
    lax.fori_loop(0, ECH, chunk, 0)
    plsc.subcore_barrier()
    pltpu.sync_copy(acc.at[pl.ds(s * 256, 256)],
                    agg_hbm.at[c, pl.ds(s * 256, 256)])


def _edge_mp(rows, cols, hr, hc, he):
    mesh = plsc.VectorSubcoreMesh(core_axis_name="c", subcore_axis_name="s",
                                  num_cores=2, num_subcores=16)
    f = pl.kernel(
        _edge_mp_body,
        out_type=jax.ShapeDtypeStruct((2, N, HIDP), jnp.float32),
        mesh=mesh,
        scratch_types=[
            pltpu.VMEM((ECH, 128), jnp.int32),
            pltpu.VMEM((ECH, 128), jnp.int32),
            pltpu.VMEM((128, HIDP), jnp.float32),
            pltpu.VMEM((128, HIDP), jnp.float32),
            pltpu.VMEM((128, HIDP), jnp.float32),
            pltpu.VMEM((256, HIDP), jnp.float32),
            pltpu.VMEM_SHARED((N, HIDP), jnp.float32),
        ],
    )
    return f(rows, cols, hr, hc, he)


def _spmm_body(a_ref, h_ref, w_ref, o_ref, *, relu):
    s = jnp.dot(a_ref[...], h_ref[...], preferred_element_type=jnp.float32)
    if relu:
        s = jnp.maximum(s, 0.0)
    if w_ref is not None:
        s = jnp.dot(s, w_ref[...], preferred_element_type=jnp.float32)
    o_ref[...] = s


def _spmm_mm(a, h, w=None, relu=False):
    d = h.shape[1]
    dout = d if w is None else w.shape[1]
    if w is None:
        body = functools.partial(lambda ar, hr, orr, relu: _spmm_body(ar, hr, None, orr, relu=relu), relu=relu)
        in_specs = [_rows(RB, N), _full((N, d))]
        args = (a, h)
    else:
        body = functools.partial(_spmm_body, relu=relu)
        in_specs = [_rows(RB, N), _full((N, d)), _full(w.shape)]
        args = (a, h, w)
    return pl.pallas_call(
        body,
        grid=(N // RB,),
        in_specs=in_specs,
        out_specs=_rows(RB, dout),
        out_shape=jax.ShapeDtypeStruct((N, dout), jnp.float32),
    )(*args)


def _ae_enc_body(x_ref, w0, b0, w1, b1, w2, b2, w3, b3, wg0, zae_ref, g0_ref):
    h = x_ref[...]
    g0_ref[...] = jnp.dot(h, wg0[...], preferred_element_type=jnp.float32)
    h1 = jnp.maximum(jnp.dot(h, w0[...], preferred_element_type=jnp.float32) + b0[...], 0.0)
    h2 = jnp.maximum(jnp.dot(h1, w1[...], preferred_element_type=jnp.float32) + b1[...], 0.0)
    h3 = jnp.maximum(jnp.dot(h2, w2[...], preferred_element_type=jnp.float32) + b2[...], 0.0)
    zae_ref[...] = jnp.dot(h3, w3[...], preferred_element_type=jnp.float32) + b3[...]


def _ae_encode(x, p):
    ws = []
    specs = [_rows(RB, N_INPUT)]
    for i in range(4):
        w = p['ae_enc_W%d' % i]
        b = p['ae_enc_b%d' % i].reshape(1, -1)
        ws += [w, b]
        specs += [_full(w.shape), _full(b.shape)]
    ws.append(p['gae_enc_W0'])
    specs.append(_full(p['gae_enc_W0'].shape))
    return pl.pallas_call(
        _ae_enc_body,
        grid=(N // RB,),
        in_specs=specs,
        out_specs=[_rows(RB, N_Z), _rows(RB, 128)],
        out_shape=[jax.ShapeDtypeStruct((N, N_Z), jnp.float32),
                   jax.ShapeDtypeStruct((N, 128), jnp.float32)],
    )(x, *ws)


def _pad_lanes(v):
    return jnp.concatenate([v, jnp.zeros_like(v)], axis=1)


def _comb_body(zae_ref, zegae_ref, a_ref, wv, bv, wra, wrb,
               zi_ref, hv_ref, hr_ref, hc_ref):
    a = a_ref[...]
    zi = a * zae_ref[...] + (1.0 - a) * zegae_ref[...]
    zi_ref[...] = zi
    hv = jnp.maximum(jnp.dot(zi, wv[...], preferred_element_type=jnp.float32) + bv[...], 0.0)
    hv_ref[...] = hv
    hr_ref[...] = _pad_lanes(jnp.dot(hv, wra[...], preferred_element_type=jnp.float32))
    hc_ref[...] = _pad_lanes(jnp.dot(hv, wrb[...], preferred_element_type=jnp.float32))


def _combine_hv(z_ae, z_egae, p):
    wm = p['gl_Wm0']
    wra, wrb = wm[:HID], wm[HID:2 * HID]
    args = (z_ae, z_egae, p['a'], p['gl_Wv'], p['gl_bv'].reshape(1, -1), wra, wrb)
    specs = [_rows(RB, N_Z), _rows(RB, N_Z), _rows(RB, N_Z),
             _full(args[3].shape), _full(args[4].shape), _full(wra.shape), _full(wrb.shape)]
    return pl.pallas_call(
        _comb_body,
        grid=(N // RB,),
        in_specs=specs,
        out_specs=[_rows(RB, N_Z), _rows(RB, HID), _rows(RB, HIDP), _rows(RB, HIDP)],
        out_shape=[jax.ShapeDtypeStruct((N, N_Z), jnp.float32),
                   jax.ShapeDtypeStruct((N, HID), jnp.float32),
                   jax.ShapeDtypeStruct((N, HIDP), jnp.float32),
                   jax.ShapeDtypeStruct((N, HIDP), jnp.float32)],
    )(*args)


def _he_body(ef_ref, we, be, wc0, bm0, wc1, bm1, he0_ref, he1_ref):
    he = jnp.maximum(ef_ref[...] * we[...] + be[...], 0.0)
    he0_ref[...] = _pad_lanes(jnp.dot(he, wc0[...], preferred_element_type=jnp.float32) + bm0[...])
    he1_ref[...] = _pad_lanes(jnp.dot(he, wc1[...], preferred_element_type=jnp.float32) + bm1[...])


def _he_tables(ef, p):
    args = (ef.reshape(E, 1), p['gl_We'], p['gl_be'].reshape(1, -1),
            p['gl_Wm0'][2 * HID:], p['gl_bm0'].reshape(1, -1),
            p['gl_Wm1'][2 * HID:], p['gl_bm1'].reshape(1, -1))
    specs = [_rows(EB, 1)] + [_full(a.shape) for a in args[1:]]
    return pl.pallas_call(
        _he_body,
        grid=(E // EB,),
        in_specs=specs,
        out_specs=[_rows(EB, HIDP)] * 2,
        out_shape=[jax.ShapeDtypeStruct((E, HIDP), jnp.float32)] * 2,
    )(*args)


def _upd_body(hv_ref, agga_ref, aggb_ref, wu, bu, wra, wrb, hv1_ref, hr_ref, hc_ref):
    agg = (agga_ref[0] + aggb_ref[0])[:, :HID]
    hv1 = jnp.maximum(
        hv_ref[...] + jnp.dot(agg, wu[...], preferred_element_type=jnp.float32)
        + bu[...], 0.0)
    hv1_ref[...] = hv1
    hr_ref[...] = _pad_lanes(jnp.dot(hv1, wra[...], preferred_element_type=jnp.float32))
    hc_ref[...] = _pad_lanes(jnp.dot(hv1, wrb[...], preferred_element_type=jnp.float32))


def _agg_spec():
    return pl.BlockSpec((1, RB, HIDP), lambda i: (0, i, 0))


def _hv_update(hv, aggp, p):
    wm = p['gl_Wm1']
    wra, wrb = wm[:HID], wm[HID:2 * HID]
    args = (hv, aggp[0:1], aggp[1:2], p['gl_Wu0'], p['gl_bu0'].reshape(1, -1), wra, wrb)
    specs = [_rows(RB, HID), _agg_spec(), _agg_spec()] + [_full(a.shape) for a in args[3:]]
    return pl.pallas_call(
        _upd_body,
        grid=(N // RB,),
        in_specs=specs,
        out_specs=[_rows(RB, HID), _rows(RB, HIDP), _rows(RB, HIDP)],
        out_shape=[jax.ShapeDtypeStruct((N, HID), jnp.float32),
                   jax.ShapeDtypeStruct((N, HIDP), jnp.float32),
                   jax.ShapeDtypeStruct((N, HIDP), jnp.float32)],
    )(*args)


def _fin_body(hv_ref, agga_ref, aggb_ref, zl_ref, wu, bu, wout, bout, gamma,
              w0, b0, w1, b1, w2, b2, w3, b3, wg0,
              zt_ref, xhat_ref, u0_ref):
    agg = (agga_ref[0] + aggb_ref[0])[:, :HID]
    hv2 = jnp.maximum(
        hv_ref[...] + jnp.dot(agg, wu[...], preferred_element_type=jnp.float32)
        + bu[...], 0.0)
    zg = jnp.dot(hv2, wout[...], preferred_element_type=jnp.float32) + bout[...]
    zt = gamma[...] * zg + zl_ref[...]
    zt_ref[...] = zt
    u0_ref[...] = jnp.dot(zt, wg0[...], preferred_element_type=jnp.float32)
    h1 = jnp.maximum(jnp.dot(zt, w0[...], preferred_element_type=jnp.float32) + b0[...], 0.0)
    h2 = jnp.maximum(jnp.dot(h1, w1[...], preferred_element_type=jnp.float32) + b1[...], 0.0)
    h3 = jnp.maximum(jnp.dot(h2, w2[...], preferred_element_type=jnp.float32) + b2[...], 0.0)
    xhat_ref[...] = jnp.dot(h3, w3[...], preferred_element_type=jnp.float32) + b3[...]


def _final_update(hv1, aggp1, z_l, p):
    args = [hv1, aggp1[0:1], aggp1[1:2], z_l, p['gl_Wu1'], p['gl_bu1'].reshape(1, -1),
            p['gl_Wout'], p['gl_bout'].reshape(1, -1), p['gamma'].reshape(1, 1)]
    for i in range(4):
        args += [p['ae_dec_W%d' % i], p['ae_dec_b%d' % i].reshape(1, -1)]
    args.append(p['gae_dec_W0'])
    specs = [_rows(RB, HID), _agg_spec(), _agg_spec(), _rows(RB, N_Z)] \
        + [_full(a.shape) for a in args[4:]]
    return pl.pallas_call(
        _fin_body,
        grid=(N // RB,),
        in_specs=specs,
        out_specs=[_rows(RB, N_Z), _rows(RB, N_INPUT), _rows(RB, 256)],
        out_shape=[jax.ShapeDtypeStruct((N, N_Z), jnp.float32),
                   jax.ShapeDtypeStruct((N, N_INPUT), jnp.float32),
                   jax.ShapeDtypeStruct((N, 256), jnp.float32)],
    )(*args)


AB = 512


def _adj_body(zi_ref, zj_ref, hi_ref, hj_ref, o_ref):
    s1 = jax.lax.dot_general(zi_ref[...], zj_ref[...],
                             (((1,), (1,)), ((), ())),
                             preferred_element_type=jnp.float32)
    s2 = jax.lax.dot_general(hi_ref[...], hj_ref[...],
                             (((1,), (1,)), ((), ())),
                             preferred_element_type=jnp.float32)
    o_ref[...] = jax.nn.sigmoid(s1) + jax.nn.sigmoid(s2)


def _adj_hat(z_egae, z_hat):
    return pl.pallas_call(
        _adj_body,
        grid=(N // AB, N // AB),
        in_specs=[pl.BlockSpec((AB, N_Z), lambda i, j: (i, 0)),
                  pl.BlockSpec((AB, N_Z), lambda i, j: (j, 0)),
                  pl.BlockSpec((AB, N_INPUT), lambda i, j: (i, 0)),
                  pl.BlockSpec((AB, N_INPUT), lambda i, j: (j, 0))],
        out_specs=pl.BlockSpec((AB, AB), lambda i, j: (i, j)),
        out_shape=jax.ShapeDtypeStruct((N, N), jnp.float32),
    )(z_egae, z_egae, z_hat, z_hat)


def _soft_body(z1_ref, z2_ref, z3_ref, cl_ref, cn_ref, q1_ref, q2_ref, q3_ref):
    cl = cl_ref[...]
    cn = cn_ref[...]
    for z_ref, q_ref in ((z1_ref, q1_ref), (z2_ref, q2_ref), (z3_ref, q3_ref)):
        z = z_ref[...]
        zn = jnp.sum(z * z, axis=1, keepdims=True)
        d = zn + cn - 2.0 * jax.lax.dot_general(
            z, cl, (((1,), (1,)), ((), ())), preferred_element_type=jnp.float32)
        q = 1.0 / (1.0 + d / V)
        q_ref[...] = q / jnp.sum(q, axis=1, keepdims=True)


def _softs(z_tilde, z_ae, z_egae, cl):
    cn = jnp.sum(cl * cl, axis=1).reshape(1, -1)
    nc = cl.shape[0]
    return pl.pallas_call(
        _soft_body,
        grid=(N // AB,),
        in_specs=[_rows(AB, N_Z)] * 3 + [_full(cl.shape), _full(cn.shape)],
        out_specs=[_rows(AB, nc)] * 3,
        out_shape=[jax.ShapeDtypeStruct((N, nc), jnp.float32)] * 3,
    )(z_tilde, z_ae, z_egae, cl, cn)


def kernel(x, adj_values, distance, params, edge_index):
    p = params
    rows = edge_index[0].astype(jnp.int32)
    cols = edge_index[1].astype(jnp.int32)

    a_flat, ef = _sc_build(rows, cols, adj_values, distance.reshape(-1))
    A = a_flat.reshape(N, N)

    z_ae, g0 = _ae_encode(x, p)

    g1 = _spmm_mm(A, g0, p['gae_enc_W1'], relu=True)
    g2 = _spmm_mm(A, g1, p['gae_enc_W2'], relu=True)
    z_egae = _spmm_mm(A, g2)

    z_i, hv, hr0, hc0 = _combine_hv(z_ae, z_egae, p)
    z_l = _spmm_mm(A, z_i)

    he0, he1 = _he_tables(ef, p)

    aggp0 = _edge_mp(rows, cols, hr0, hc0, he0)
    hv1, hr1, hc1 = _hv_update(hv, aggp0, p)

    aggp1 = _edge_mp(rows, cols, hr1, hc1, he1)

    z_tilde, x_hat, u0 = _final_update(hv1, aggp1, z_l, p)

    u1 = _spmm_mm(A, u0, p['gae_dec_W1'], relu=True)
    u2 = _spmm_mm(A, u1, p['gae_dec_W2'], relu=True)
    z_hat = _spmm_mm(A, u2)

    adj_hat = _adj_hat(z_egae, z_hat)
    q1, q2, q3 = _softs(z_tilde, z_ae, z_egae, p['cluster'])

    return (x_hat, z_hat, adj_hat, z_ae, z_egae, q1, q2, q3, z_tilde)


# speedup vs baseline: 5.5654x; 1.0346x over previous
import functools

import jax
import jax.numpy as jnp
from jax import lax
from jax.experimental import pallas as pl
from jax.experimental.pallas import tpu as pltpu
from jax.experimental.pallas import tpu_sc as plsc

N = 4096
E = 65536
N_INPUT = 512
N_Z = 20
HID = 64
V = 1.0

RB = 256
EB = 1024


def _full(shape):
    return pl.BlockSpec(shape, lambda *_: (0,) * len(shape))


def _rows(bs, ncols):
    return pl.BlockSpec((bs, ncols), lambda i: (i, 0))


GR = 256
NG = N // GR
ACC_LEN = (GR + 2) * N
DUMP = GR * N
EPT = E // 16
EFT = E // 32
ZB = 16384
TPW = ACC_LEN // 16


NCH = EPT // 128


def _abuild_body(rows_hbm, cols_hbm, vals_hbm, dist_hbm, a_hbm, ef_hbm,
                 rows_v, cols_v, vals_v, lin_v, idx_v, ef_v, zbuf, acc):
    c = lax.axis_index("c")
    s = lax.axis_index("s")
    ebase = s * EPT
    pltpu.sync_copy(rows_hbm.at[pl.ds(ebase, EPT)], rows_v)
    pltpu.sync_copy(cols_hbm.at[pl.ds(ebase, EPT)], cols_v)
    pltpu.sync_copy(vals_hbm.at[pl.ds(ebase, EPT)], vals_v)

    def zb_init(i, _):
        zbuf[pl.ds(i * 16, 16)] = jnp.zeros((16,), jnp.float32)
        return 0
    lax.fori_loop(0, ZB // 16, zb_init, 0)

    def lin_init(i, _):
        o = i * 16
        lin_v[pl.ds(o, 16)] = rows_v[pl.ds(o, 16)] * N + cols_v[pl.ds(o, 16)]
        return 0
    lax.fori_loop(0, EPT // 16, lin_init, 0)

    off0 = c * EFT

    def ef_chunk(j, _):
        def lanes(l, _):
            o = j * 128 + l * 16
            row = idx_v.at[j]
            row[pl.ds(l * 16, 16)] = lin_v[pl.ds(off0 + o, 16)]
            return 0
        lax.fori_loop(0, 8, lanes, 0)
        pltpu.sync_copy(dist_hbm.at[idx_v.at[j]], ef_v)
        pltpu.sync_copy(ef_v, ef_hbm.at[pl.ds(ebase + off0 + j * 128, 128)])
        return 0
    lax.fori_loop(0, EFT // 128, ef_chunk, 0)

    def group(g, _):
        lo = (g * 2 + c) * GR
        base = s * TPW
        for t in range(4):
            pltpu.sync_copy(zbuf, acc.at[pl.ds(base + t * ZB, ZB)])
        pltpu.sync_copy(zbuf.at[pl.ds(0, TPW - 4 * ZB)],
                        acc.at[pl.ds(base + 4 * ZB, TPW - 4 * ZB)])
        plsc.subcore_barrier()

        def chunk(j, _):
            def lanes(l, _):
                o = j * 128 + l * 16
                gl = lin_v[pl.ds(o, 16)]
                rel = gl - lo * N
                ing = (rel >= 0) & (rel < GR * N)
                row = idx_v.at[j]
                row[pl.ds(l * 16, 16)] = jnp.where(
                    ing, rel, DUMP + (gl & (N - 1)))
                return 0
            lax.fori_loop(0, 8, lanes, 0)
            pltpu.sync_copy(vals_v.at[pl.ds(j * 128, 128)],
                            acc.at[idx_v.at[j]], add=True)
            return 0
        lax.fori_loop(0, NCH, chunk, 0)
        plsc.subcore_barrier()

        def flush(r, _):
            pltpu.sync_copy(acc.at[pl.ds((s * 16 + r) * N, N)],
                            a_hbm.at[lo + s * 16 + r])
            return 0
        lax.fori_loop(0, 16, flush, 0)
        plsc.subcore_barrier()
        return 0
    lax.fori_loop(0, NG // 2, group, 0)


def _sc_build(rows, cols, vals, dist_flat):
    mesh = plsc.VectorSubcoreMesh(core_axis_name="c", subcore_axis_name="s",
                                  num_cores=2, num_subcores=16)
    f = pl.kernel(
        _abuild_body,
        out_type=[jax.ShapeDtypeStruct((N, N), jnp.float32),
                  jax.ShapeDtypeStruct((E,), jnp.float32)],
        mesh=mesh,
        scratch_types=[
            pltpu.VMEM((EPT,), jnp.int32),
            pltpu.VMEM((EPT,), jnp.int32),
            pltpu.VMEM((EPT,), jnp.float32),
            pltpu.VMEM((EPT,), jnp.int32),
            pltpu.VMEM((NCH, 128), jnp.int32),
            pltpu.VMEM((128,), jnp.float32),
            pltpu.VMEM((ZB,), jnp.float32),
            pltpu.VMEM_SHARED((ACC_LEN,), jnp.float32),
        ],
    )
    return f(rows, cols, vals, dist_flat)


EMT = E // 32
ECH = EMT // 128


HIDP = 128


def _edge_mp_body(rows_hbm, cols_hbm, hr_hbm, hc_hbm, he_hbm, agg_hbm,
                  ridx, cidx, ga, gb, mb, zbuf, acc):
    c = lax.axis_index("c")
    s = lax.axis_index("s")
    tid = s * 2 + c
    ebase = tid * EMT

    def ld(j, _):
        pltpu.sync_copy(rows_hbm.at[pl.ds(ebase + j * 128, 128)], ridx.at[j])
        pltpu.sync_copy(cols_hbm.at[pl.ds(ebase + j * 128, 128)], cidx.at[j])
        return 0
    lax.fori_loop(0, ECH, ld, 0)

    def zb_init(i, _):
        def inner(k, _):
            zbuf[i, pl.ds(k * 16, 16)] = jnp.zeros((16,), jnp.float32)
            return 0
        lax.fori_loop(0, HIDP // 16, inner, 0)
        return 0
    lax.fori_loop(0, 256, zb_init, 0)
    pltpu.sync_copy(zbuf, acc.at[pl.ds(s * 256, 256)])
    plsc.subcore_barrier()

    def chunk(j, _):
        pltpu.sync_copy(hr_hbm.at[ridx.at[j]], ga)
        pltpu.sync_copy(hc_hbm.at[cidx.at[j]], gb)
        pltpu.sync_copy(he_hbm.at[pl.ds(ebase + j * 128, 128)], mb)

        def row(i, _):
            for k in range(HID // 16):
                d = pl.ds(k * 16, 16)
                mb[i, d] = jnp.maximum(ga[i, d] + gb[i, d] + mb[i, d], 0.0)
            return 0
        lax.fori_loop(0, 128, row, 0)
        pltpu.sync_copy(mb, acc.at[cidx.at[j]], add=True)
        return 0
    lax.fori_loop(0, ECH, chunk, 0)
    plsc.subcore_barrier()
    pltpu.sync_copy(acc.at[pl.ds(s * 256, 256)],
                    agg_hbm.at[c, pl.ds(s * 256, 256)])


def _edge_mp(rows, cols, hr, hc, he):
    mesh = plsc.VectorSubcoreMesh(core_axis_name="c", subcore_axis_name="s",
                                  num_cores=2, num_subcores=16)
    f = pl.kernel(
        _edge_mp_body,
        out_type=jax.ShapeDtypeStruct((2, N, HIDP), jnp.float32),
        mesh=mesh,
        scratch_types=[
            pltpu.VMEM((ECH, 128), jnp.int32),
            pltpu.VMEM((ECH, 128), jnp.int32),
            pltpu.VMEM((128, HIDP), jnp.float32),
            pltpu.VMEM((128, HIDP), jnp.float32),
            pltpu.VMEM((128, HIDP), jnp.float32),
            pltpu.VMEM((256, HIDP), jnp.float32),
            pltpu.VMEM_SHARED((N, HIDP), jnp.float32),
        ],
    )
    return f(rows, cols, hr, hc, he)


def _spmm_body(a_ref, h_ref, w_ref, o_ref, *, relu):
    s = jnp.dot(a_ref[...].astype(jnp.bfloat16), h_ref[...].astype(jnp.bfloat16),
                preferred_element_type=jnp.float32)
    if relu:
        s = jnp.maximum(s, 0.0)
    if w_ref is not None:
        s = jnp.dot(s, w_ref[...], preferred_element_type=jnp.float32)
    o_ref[...] = s


def _spmm_mm(a, h, w=None, relu=False):
    d = h.shape[1]
    dout = d if w is None else w.shape[1]
    if w is None:
        body = functools.partial(lambda ar, hr, orr, relu: _spmm_body(ar, hr, None, orr, relu=relu), relu=relu)
        in_specs = [_rows(RB, N), _full((N, d))]
        args = (a, h)
    else:
        body = functools.partial(_spmm_body, relu=relu)
        in_specs = [_rows(RB, N), _full((N, d)), _full(w.shape)]
        args = (a, h, w)
    return pl.pallas_call(
        body,
        grid=(N // RB,),
        in_specs=in_specs,
        out_specs=_rows(RB, dout),
        out_shape=jax.ShapeDtypeStruct((N, dout), jnp.float32),
    )(*args)


def _ae_enc_body(x_ref, w0, b0, w1, b1, w2, b2, w3, b3, wg0, zae_ref, g0_ref):
    h = x_ref[...]
    g0_ref[...] = jnp.dot(h, wg0[...], preferred_element_type=jnp.float32)
    h1 = jnp.maximum(jnp.dot(h, w0[...], preferred_element_type=jnp.float32) + b0[...], 0.0)
    h2 = jnp.maximum(jnp.dot(h1, w1[...], preferred_element_type=jnp.float32) + b1[...], 0.0)
    h3 = jnp.maximum(jnp.dot(h2, w2[...], preferred_element_type=jnp.float32) + b2[...], 0.0)
    zae_ref[...] = jnp.dot(h3, w3[...], preferred_element_type=jnp.float32) + b3[...]


def _ae_encode(x, p):
    ws = []
    specs = [_rows(RB, N_INPUT)]
    for i in range(4):
        w = p['ae_enc_W%d' % i]
        b = p['ae_enc_b%d' % i].reshape(1, -1)
        ws += [w, b]
        specs += [_full(w.shape), _full(b.shape)]
    ws.append(p['gae_enc_W0'])
    specs.append(_full(p['gae_enc_W0'].shape))
    return pl.pallas_call(
        _ae_enc_body,
        grid=(N // RB,),
        in_specs=specs,
        out_specs=[_rows(RB, N_Z), _rows(RB, 128)],
        out_shape=[jax.ShapeDtypeStruct((N, N_Z), jnp.float32),
                   jax.ShapeDtypeStruct((N, 128), jnp.float32)],
    )(x, *ws)


def _pad_lanes(v):
    return jnp.concatenate([v, jnp.zeros_like(v)], axis=1)


def _comb_body(zae_ref, zegae_ref, a_ref, wv, bv, wra, wrb,
               zi_ref, hv_ref, hr_ref, hc_ref):
    a = a_ref[...]
    zi = a * zae_ref[...] + (1.0 - a) * zegae_ref[...]
    zi_ref[...] = zi
    hv = jnp.maximum(jnp.dot(zi, wv[...], preferred_element_type=jnp.float32) + bv[...], 0.0)
    hv_ref[...] = hv
    hr_ref[...] = _pad_lanes(jnp.dot(hv, wra[...], preferred_element_type=jnp.float32))
    hc_ref[...] = _pad_lanes(jnp.dot(hv, wrb[...], preferred_element_type=jnp.float32))


def _combine_hv(z_ae, z_egae, p):
    wm = p['gl_Wm0']
    wra, wrb = wm[:HID], wm[HID:2 * HID]
    args = (z_ae, z_egae, p['a'], p['gl_Wv'], p['gl_bv'].reshape(1, -1), wra, wrb)
    specs = [_rows(RB, N_Z), _rows(RB, N_Z), _rows(RB, N_Z),
             _full(args[3].shape), _full(args[4].shape), _full(wra.shape), _full(wrb.shape)]
    return pl.pallas_call(
        _comb_body,
        grid=(N // RB,),
        in_specs=specs,
        out_specs=[_rows(RB, N_Z), _rows(RB, HID), _rows(RB, HIDP), _rows(RB, HIDP)],
        out_shape=[jax.ShapeDtypeStruct((N, N_Z), jnp.float32),
                   jax.ShapeDtypeStruct((N, HID), jnp.float32),
                   jax.ShapeDtypeStruct((N, HIDP), jnp.float32),
                   jax.ShapeDtypeStruct((N, HIDP), jnp.float32)],
    )(*args)


def _he_body(ef_ref, we, be, wc0, bm0, wc1, bm1, he0_ref, he1_ref):
    he = jnp.maximum(ef_ref[...] * we[...] + be[...], 0.0)
    he0_ref[...] = _pad_lanes(jnp.dot(he, wc0[...], preferred_element_type=jnp.float32) + bm0[...])
    he1_ref[...] = _pad_lanes(jnp.dot(he, wc1[...], preferred_element_type=jnp.float32) + bm1[...])


def _he_tables(ef, p):
    args = (ef.reshape(E, 1), p['gl_We'], p['gl_be'].reshape(1, -1),
            p['gl_Wm0'][2 * HID:], p['gl_bm0'].reshape(1, -1),
            p['gl_Wm1'][2 * HID:], p['gl_bm1'].reshape(1, -1))
    specs = [_rows(EB, 1)] + [_full(a.shape) for a in args[1:]]
    return pl.pallas_call(
        _he_body,
        grid=(E // EB,),
        in_specs=specs,
        out_specs=[_rows(EB, HIDP)] * 2,
        out_shape=[jax.ShapeDtypeStruct((E, HIDP), jnp.float32)] * 2,
    )(*args)


def _upd_body(hv_ref, agga_ref, aggb_ref, wu, bu, wra, wrb, hv1_ref, hr_ref, hc_ref):
    agg = (agga_ref[0] + aggb_ref[0])[:, :HID]
    hv1 = jnp.maximum(
        hv_ref[...] + jnp.dot(agg, wu[...], preferred_element_type=jnp.float32)
        + bu[...], 0.0)
    hv1_ref[...] = hv1
    hr_ref[...] = _pad_lanes(jnp.dot(hv1, wra[...], preferred_element_type=jnp.float32))
    hc_ref[...] = _pad_lanes(jnp.dot(hv1, wrb[...], preferred_element_type=jnp.float32))


def _agg_spec():
    return pl.BlockSpec((1, RB, HIDP), lambda i: (0, i, 0))


def _hv_update(hv, aggp, p):
    wm = p['gl_Wm1']
    wra, wrb = wm[:HID], wm[HID:2 * HID]
    args = (hv, aggp[0:1], aggp[1:2], p['gl_Wu0'], p['gl_bu0'].reshape(1, -1), wra, wrb)
    specs = [_rows(RB, HID), _agg_spec(), _agg_spec()] + [_full(a.shape) for a in args[3:]]
    return pl.pallas_call(
        _upd_body,
        grid=(N // RB,),
        in_specs=specs,
        out_specs=[_rows(RB, HID), _rows(RB, HIDP), _rows(RB, HIDP)],
        out_shape=[jax.ShapeDtypeStruct((N, HID), jnp.float32),
                   jax.ShapeDtypeStruct((N, HIDP), jnp.float32),
                   jax.ShapeDtypeStruct((N, HIDP), jnp.float32)],
    )(*args)


def _fin_body(hv_ref, agga_ref, aggb_ref, zl_ref, wu, bu, wout, bout, gamma,
              w0, b0, w1, b1, w2, b2, w3, b3, wg0,
              zt_ref, xhat_ref, u0_ref):
    agg = (agga_ref[0] + aggb_ref[0])[:, :HID]
    hv2 = jnp.maximum(
        hv_ref[...] + jnp.dot(agg, wu[...], preferred_element_type=jnp.float32)
        + bu[...], 0.0)
    zg = jnp.dot(hv2, wout[...], preferred_element_type=jnp.float32) + bout[...]
    zt = gamma[...] * zg + zl_ref[...]
    zt_ref[...] = zt
    u0_ref[...] = jnp.dot(zt, wg0[...], preferred_element_type=jnp.float32)
    h1 = jnp.maximum(jnp.dot(zt, w0[...], preferred_element_type=jnp.float32) + b0[...], 0.0)
    h2 = jnp.maximum(jnp.dot(h1, w1[...], preferred_element_type=jnp.float32) + b1[...], 0.0)
    h3 = jnp.maximum(jnp.dot(h2, w2[...], preferred_element_type=jnp.float32) + b2[...], 0.0)
    xhat_ref[...] = jnp.dot(h3, w3[...], preferred_element_type=jnp.float32) + b3[...]


def _final_update(hv1, aggp1, z_l, p):
    args = [hv1, aggp1[0:1], aggp1[1:2], z_l, p['gl_Wu1'], p['gl_bu1'].reshape(1, -1),
            p['gl_Wout'], p['gl_bout'].reshape(1, -1), p['gamma'].reshape(1, 1)]
    for i in range(4):
        args += [p['ae_dec_W%d' % i], p['ae_dec_b%d' % i].reshape(1, -1)]
    args.append(p['gae_dec_W0'])
    specs = [_rows(RB, HID), _agg_spec(), _agg_spec(), _rows(RB, N_Z)] \
        + [_full(a.shape) for a in args[4:]]
    return pl.pallas_call(
        _fin_body,
        grid=(N // RB,),
        in_specs=specs,
        out_specs=[_rows(RB, N_Z), _rows(RB, N_INPUT), _rows(RB, 256)],
        out_shape=[jax.ShapeDtypeStruct((N, N_Z), jnp.float32),
                   jax.ShapeDtypeStruct((N, N_INPUT), jnp.float32),
                   jax.ShapeDtypeStruct((N, 256), jnp.float32)],
    )(*args)


AB = 512


def _adj_body(zi_ref, zj_ref, hi_ref, hj_ref, o_ref):
    s1 = jax.lax.dot_general(zi_ref[...], zj_ref[...],
                             (((1,), (1,)), ((), ())),
                             preferred_element_type=jnp.float32)
    s2 = jax.lax.dot_general(hi_ref[...].astype(jnp.bfloat16),
                             hj_ref[...].astype(jnp.bfloat16),
                             (((1,), (1,)), ((), ())),
                             preferred_element_type=jnp.float32)
    o_ref[...] = jax.nn.sigmoid(s1) + jax.nn.sigmoid(s2)


def _adj_hat(z_egae, z_hat):
    return pl.pallas_call(
        _adj_body,
        grid=(N // AB, N // AB),
        in_specs=[pl.BlockSpec((AB, N_Z), lambda i, j: (i, 0)),
                  pl.BlockSpec((AB, N_Z), lambda i, j: (j, 0)),
                  pl.BlockSpec((AB, N_INPUT), lambda i, j: (i, 0)),
                  pl.BlockSpec((AB, N_INPUT), lambda i, j: (j, 0))],
        out_specs=pl.BlockSpec((AB, AB), lambda i, j: (i, j)),
        out_shape=jax.ShapeDtypeStruct((N, N), jnp.float32),
    )(z_egae, z_egae, z_hat, z_hat)


def _soft_body(z1_ref, z2_ref, z3_ref, cl_ref, cn_ref, q1_ref, q2_ref, q3_ref):
    cl = cl_ref[...]
    cn = cn_ref[...]
    for z_ref, q_ref in ((z1_ref, q1_ref), (z2_ref, q2_ref), (z3_ref, q3_ref)):
        z = z_ref[...]
        zn = jnp.sum(z * z, axis=1, keepdims=True)
        d = zn + cn - 2.0 * jax.lax.dot_general(
            z, cl, (((1,), (1,)), ((), ())), preferred_element_type=jnp.float32)
        q = 1.0 / (1.0 + d / V)
        q_ref[...] = q / jnp.sum(q, axis=1, keepdims=True)


def _softs(z_tilde, z_ae, z_egae, cl):
    cn = jnp.sum(cl * cl, axis=1).reshape(1, -1)
    nc = cl.shape[0]
    return pl.pallas_call(
        _soft_body,
        grid=(N // AB,),
        in_specs=[_rows(AB, N_Z)] * 3 + [_full(cl.shape), _full(cn.shape)],
        out_specs=[_rows(AB, nc)] * 3,
        out_shape=[jax.ShapeDtypeStruct((N, nc), jnp.float32)] * 3,
    )(z_tilde, z_ae, z_egae, cl, cn)


def kernel(x, adj_values, distance, params, edge_index):
    p = params
    rows = edge_index[0].astype(jnp.int32)
    cols = edge_index[1].astype(jnp.int32)

    a_flat, ef = _sc_build(rows, cols, adj_values, distance.reshape(-1))
    A = a_flat.reshape(N, N)

    z_ae, g0 = _ae_encode(x, p)

    g1 = _spmm_mm(A, g0, p['gae_enc_W1'], relu=True)
    g2 = _spmm_mm(A, g1, p['gae_enc_W2'], relu=True)
    z_egae = _spmm_mm(A, g2)

    z_i, hv, hr0, hc0 = _combine_hv(z_ae, z_egae, p)
    z_l = _spmm_mm(A, z_i)

    he0, he1 = _he_tables(ef, p)

    aggp0 = _edge_mp(rows, cols, hr0, hc0, he0)
    hv1, hr1, hc1 = _hv_update(hv, aggp0, p)

    aggp1 = _edge_mp(rows, cols, hr1, hc1, he1)

    z_tilde, x_hat, u0 = _final_update(hv1, aggp1, z_l, p)

    u1 = _spmm_mm(A, u0, p['gae_dec_W1'], relu=True)
    u2 = _spmm_mm(A, u1, p['gae_dec_W2'], relu=True)
    z_hat = _spmm_mm(A, u2)

    adj_hat = _adj_hat(z_egae, z_hat)
    q1, q2, q3 = _softs(z_tilde, z_ae, z_egae, p['cluster'])

    return (x_hat, z_hat, adj_hat, z_ae, z_egae, q1, q2, q3, z_tilde)


# speedup vs baseline: 5.6853x; 1.0215x over previous
import functools

import jax
import jax.numpy as jnp
from jax import lax
from jax.experimental import pallas as pl
from jax.experimental.pallas import tpu as pltpu
from jax.experimental.pallas import tpu_sc as plsc

N = 4096
E = 65536
N_INPUT = 512
N_Z = 20
HID = 64
V = 1.0

RB = 256
EB = 1024


def _full(shape):
    return pl.BlockSpec(shape, lambda *_: (0,) * len(shape))


def _rows(bs, ncols):
    return pl.BlockSpec((bs, ncols), lambda i: (i, 0))


GR = 64
NG = N // GR
ACC_LEN = (GR + 2) * N
DUMP = GR * N
FRT = GR // 16
EPT = E // 16
EFT = E // 32
ZB = 16384
TPW = ACC_LEN // 16


NCH = EPT // 128


def _abuild_body(rows_hbm, cols_hbm, vals_hbm, dist_hbm, a_hbm, ef_hbm,
                 rows_v, cols_v, vals_v, lin_v, idx_v, ef_v, zbuf, acc,
                 ssem, fsem):
    c = lax.axis_index("c")
    s = lax.axis_index("s")
    ebase = s * EPT
    pltpu.sync_copy(rows_hbm.at[pl.ds(ebase, EPT)], rows_v)
    pltpu.sync_copy(cols_hbm.at[pl.ds(ebase, EPT)], cols_v)
    pltpu.sync_copy(vals_hbm.at[pl.ds(ebase, EPT)], vals_v)

    def zb_init(i, _):
        zbuf[pl.ds(i * 16, 16)] = jnp.zeros((16,), jnp.float32)
        return 0
    lax.fori_loop(0, ZB // 16, zb_init, 0)

    def lin_init(i, _):
        o = i * 16
        lin_v[pl.ds(o, 16)] = rows_v[pl.ds(o, 16)] * N + cols_v[pl.ds(o, 16)]
        return 0
    lax.fori_loop(0, EPT // 16, lin_init, 0)

    off0 = c * EFT

    def ef_chunk(j, _):
        def lanes(l, _):
            o = j * 128 + l * 16
            row = idx_v.at[j]
            row[pl.ds(l * 16, 16)] = lin_v[pl.ds(off0 + o, 16)]
            return 0
        lax.fori_loop(0, 8, lanes, 0)
        pltpu.sync_copy(dist_hbm.at[idx_v.at[j]], ef_v)
        pltpu.sync_copy(ef_v, ef_hbm.at[pl.ds(ebase + off0 + j * 128, 128)])
        return 0
    lax.fori_loop(0, EFT // 128, ef_chunk, 0)

    def group(g, _):
        lo = (g * 2 + c) * GR
        base = s * TPW
        for t in range(TPW // ZB):
            pltpu.sync_copy(zbuf, acc.at[pl.ds(base + t * ZB, ZB)])
        if TPW % ZB:
            pltpu.sync_copy(zbuf.at[pl.ds(0, TPW % ZB)],
                            acc.at[pl.ds(base + (TPW // ZB) * ZB, TPW % ZB)])
        plsc.subcore_barrier()

        def chunk_fill(j):
            def lanes(l, _):
                o = j * 128 + l * 16
                gl = lin_v[pl.ds(o, 16)]
                rel = gl - lo * N
                ing = (rel >= 0) & (rel < GR * N)
                row = idx_v.at[j]
                row[pl.ds(l * 16, 16)] = jnp.where(
                    ing, rel, DUMP + (gl & (N - 1)))
                return 0
            lax.fori_loop(0, 8, lanes, 0)
            return pltpu.async_copy(vals_v.at[pl.ds(j * 128, 128)],
                                    acc.at[idx_v.at[j]], ssem, add=True)

        for jb in range(0, NCH, 8):
            pend = [chunk_fill(j) for j in range(jb, jb + 8)]
            for d in pend:
                d.wait()
        plsc.subcore_barrier()

        pend = [pltpu.async_copy(acc.at[pl.ds((s * FRT + r) * N, N)],
                                 a_hbm.at[lo + s * FRT + r], fsem)
                for r in range(FRT)]
        for d in pend:
            d.wait()
        plsc.subcore_barrier()
        return 0
    lax.fori_loop(0, NG // 2, group, 0)


def _sc_build(rows, cols, vals, dist_flat):
    mesh = plsc.VectorSubcoreMesh(core_axis_name="c", subcore_axis_name="s",
                                  num_cores=2, num_subcores=16)
    f = pl.kernel(
        _abuild_body,
        out_type=[jax.ShapeDtypeStruct((N, N), jnp.float32),
                  jax.ShapeDtypeStruct((E,), jnp.float32)],
        mesh=mesh,
        scratch_types=[
            pltpu.VMEM((EPT,), jnp.int32),
            pltpu.VMEM((EPT,), jnp.int32),
            pltpu.VMEM((EPT,), jnp.float32),
            pltpu.VMEM((EPT,), jnp.int32),
            pltpu.VMEM((NCH, 128), jnp.int32),
            pltpu.VMEM((128,), jnp.float32),
            pltpu.VMEM((ZB,), jnp.float32),
            pltpu.VMEM_SHARED((ACC_LEN,), jnp.float32),
            pltpu.SemaphoreType.DMA,
            pltpu.SemaphoreType.DMA,
        ],
    )
    return f(rows, cols, vals, dist_flat)


EMT = E // 32
ECH = EMT // 128


HIDP = 128


def _edge_mp_body(rows_hbm, cols_hbm, hr_hbm, hc_hbm, he_hbm, agg_hbm,
                  ridx, cidx, ga0, ga1, gb0, gb1, mb0, mb1, m640, m641,
                  zbuf, acc, gsem0, gsem1, ssem0, ssem1):
    ga = (ga0, ga1)
    gb = (gb0, gb1)
    mb = (mb0, mb1)
    m64 = (m640, m641)
    gsem = (gsem0, gsem1)
    ssem = (ssem0, ssem1)
    c = lax.axis_index("c")
    s = lax.axis_index("s")
    tid = s * 2 + c
    ebase = tid * EMT

    def ld(j, _):
        pltpu.sync_copy(rows_hbm.at[pl.ds(ebase + j * 128, 128)], ridx.at[j])
        pltpu.sync_copy(cols_hbm.at[pl.ds(ebase + j * 128, 128)], cidx.at[j])
        return 0
    lax.fori_loop(0, ECH, ld, 0)

    def zb_init(i, _):
        def inner(k, _):
            zbuf[i, pl.ds(k * 16, 16)] = jnp.zeros((16,), jnp.float32)
            return 0
        lax.fori_loop(0, HID // 16, inner, 0)
        return 0
    lax.fori_loop(0, 128, zb_init, 0)
    pltpu.sync_copy(zbuf, acc.at[pl.ds(s * 256, 128)])
    pltpu.sync_copy(zbuf, acc.at[pl.ds(s * 256 + 128, 128)])
    plsc.subcore_barrier()

    def chunk(j, _):
        pltpu.sync_copy(hr_hbm.at[ridx.at[j]], ga[0])
        pltpu.sync_copy(hc_hbm.at[cidx.at[j]], gb[0])
        pltpu.sync_copy(he_hbm.at[pl.ds(ebase + j * 128, 128)], mb[0])

        def row(i, _):
            for k in range(HID // 16):
                d = pl.ds(k * 16, 16)
                m640[i, d] = jnp.maximum(ga0[i, d] + gb0[i, d] + mb0[i, d], 0.0)
            return 0
        lax.fori_loop(0, 128, row, 0)
        pltpu.sync_copy(m640, acc.at[cidx.at[j]], add=True)
        return 0
    lax.fori_loop(0, ECH, chunk, 0)
    plsc.subcore_barrier()
    pltpu.sync_copy(acc.at[pl.ds(s * 256, 256)],
                    agg_hbm.at[c, pl.ds(s * 256, 256)])


def _edge_mp(rows, cols, hr, hc, he):
    mesh = plsc.VectorSubcoreMesh(core_axis_name="c", subcore_axis_name="s",
                                  num_cores=2, num_subcores=16)
    f = pl.kernel(
        _edge_mp_body,
        out_type=jax.ShapeDtypeStruct((2, N, HID), jnp.float32),
        mesh=mesh,
        scratch_types=[
            pltpu.VMEM((ECH, 128), jnp.int32),
            pltpu.VMEM((ECH, 128), jnp.int32),
            pltpu.VMEM((128, HIDP), jnp.float32),
            pltpu.VMEM((128, HIDP), jnp.float32),
            pltpu.VMEM((128, HIDP), jnp.float32),
            pltpu.VMEM((128, HIDP), jnp.float32),
            pltpu.VMEM((128, HIDP), jnp.float32),
            pltpu.VMEM((128, HIDP), jnp.float32),
            pltpu.VMEM((128, HID), jnp.float32),
            pltpu.VMEM((128, HID), jnp.float32),
            pltpu.VMEM((128, HID), jnp.float32),
            pltpu.VMEM_SHARED((N, HID), jnp.float32),
            pltpu.SemaphoreType.DMA,
            pltpu.SemaphoreType.DMA,
            pltpu.SemaphoreType.DMA,
            pltpu.SemaphoreType.DMA,
        ],
    )
    return f(rows, cols, hr, hc, he)


def _spmm_body(a_ref, h_ref, w_ref, o_ref, *, relu):
    s = jnp.dot(a_ref[...].astype(jnp.bfloat16), h_ref[...].astype(jnp.bfloat16),
                preferred_element_type=jnp.float32)
    if relu:
        s = jnp.maximum(s, 0.0)
    if w_ref is not None:
        s = jnp.dot(s, w_ref[...], preferred_element_type=jnp.float32)
    o_ref[...] = s


def _spmm_mm(a, h, w=None, relu=False):
    d = h.shape[1]
    dout = d if w is None else w.shape[1]
    if w is None:
        body = functools.partial(lambda ar, hr, orr, relu: _spmm_body(ar, hr, None, orr, relu=relu), relu=relu)
        in_specs = [_rows(RB, N), _full((N, d))]
        args = (a, h)
    else:
        body = functools.partial(_spmm_body, relu=relu)
        in_specs = [_rows(RB, N), _full((N, d)), _full(w.shape)]
        args = (a, h, w)
    return pl.pallas_call(
        body,
        grid=(N // RB,),
        in_specs=in_specs,
        out_specs=_rows(RB, dout),
        out_shape=jax.ShapeDtypeStruct((N, dout), jnp.float32),
    )(*args)


def _ae_enc_body(x_ref, w0, b0, w1, b1, w2, b2, w3, b3, wg0, zae_ref, g0_ref):
    h = x_ref[...]
    g0_ref[...] = jnp.dot(h, wg0[...], preferred_element_type=jnp.float32)
    h1 = jnp.maximum(jnp.dot(h, w0[...], preferred_element_type=jnp.float32) + b0[...], 0.0)
    h2 = jnp.maximum(jnp.dot(h1, w1[...], preferred_element_type=jnp.float32) + b1[...], 0.0)
    h3 = jnp.maximum(jnp.dot(h2, w2[...], preferred_element_type=jnp.float32) + b2[...], 0.0)
    zae_ref[...] = jnp.dot(h3, w3[...], preferred_element_type=jnp.float32) + b3[...]


def _ae_encode(x, p):
    ws = []
    specs = [_rows(RB, N_INPUT)]
    for i in range(4):
        w = p['ae_enc_W%d' % i]
        b = p['ae_enc_b%d' % i].reshape(1, -1)
        ws += [w, b]
        specs += [_full(w.shape), _full(b.shape)]
    ws.append(p['gae_enc_W0'])
    specs.append(_full(p['gae_enc_W0'].shape))
    return pl.pallas_call(
        _ae_enc_body,
        grid=(N // RB,),
        in_specs=specs,
        out_specs=[_rows(RB, N_Z), _rows(RB, 128)],
        out_shape=[jax.ShapeDtypeStruct((N, N_Z), jnp.float32),
                   jax.ShapeDtypeStruct((N, 128), jnp.float32)],
    )(x, *ws)


def _pad_lanes(v):
    return jnp.concatenate([v, jnp.zeros_like(v)], axis=1)


def _comb_body(zae_ref, zegae_ref, a_ref, wv, bv, wra, wrb,
               zi_ref, hv_ref, hr_ref, hc_ref):
    a = a_ref[...]
    zi = a * zae_ref[...] + (1.0 - a) * zegae_ref[...]
    zi_ref[...] = zi
    hv = jnp.maximum(jnp.dot(zi, wv[...], preferred_element_type=jnp.float32) + bv[...], 0.0)
    hv_ref[...] = hv
    hr_ref[...] = _pad_lanes(jnp.dot(hv, wra[...], preferred_element_type=jnp.float32))
    hc_ref[...] = _pad_lanes(jnp.dot(hv, wrb[...], preferred_element_type=jnp.float32))


def _combine_hv(z_ae, z_egae, p):
    wm = p['gl_Wm0']
    wra, wrb = wm[:HID], wm[HID:2 * HID]
    args = (z_ae, z_egae, p['a'], p['gl_Wv'], p['gl_bv'].reshape(1, -1), wra, wrb)
    specs = [_rows(RB, N_Z), _rows(RB, N_Z), _rows(RB, N_Z),
             _full(args[3].shape), _full(args[4].shape), _full(wra.shape), _full(wrb.shape)]
    return pl.pallas_call(
        _comb_body,
        grid=(N // RB,),
        in_specs=specs,
        out_specs=[_rows(RB, N_Z), _rows(RB, HID), _rows(RB, HIDP), _rows(RB, HIDP)],
        out_shape=[jax.ShapeDtypeStruct((N, N_Z), jnp.float32),
                   jax.ShapeDtypeStruct((N, HID), jnp.float32),
                   jax.ShapeDtypeStruct((N, HIDP), jnp.float32),
                   jax.ShapeDtypeStruct((N, HIDP), jnp.float32)],
    )(*args)


def _he_body(ef_ref, we, be, wc0, bm0, wc1, bm1, he0_ref, he1_ref):
    he = jnp.maximum(ef_ref[...] * we[...] + be[...], 0.0)
    he0_ref[...] = _pad_lanes(jnp.dot(he, wc0[...], preferred_element_type=jnp.float32) + bm0[...])
    he1_ref[...] = _pad_lanes(jnp.dot(he, wc1[...], preferred_element_type=jnp.float32) + bm1[...])


def _he_tables(ef, p):
    args = (ef.reshape(E, 1), p['gl_We'], p['gl_be'].reshape(1, -1),
            p['gl_Wm0'][2 * HID:], p['gl_bm0'].reshape(1, -1),
            p['gl_Wm1'][2 * HID:], p['gl_bm1'].reshape(1, -1))
    specs = [_rows(EB, 1)] + [_full(a.shape) for a in args[1:]]
    return pl.pallas_call(
        _he_body,
        grid=(E // EB,),
        in_specs=specs,
        out_specs=[_rows(EB, HIDP)] * 2,
        out_shape=[jax.ShapeDtypeStruct((E, HIDP), jnp.float32)] * 2,
    )(*args)


def _upd_body(hv_ref, agga_ref, aggb_ref, wu, bu, wra, wrb, hv1_ref, hr_ref, hc_ref):
    agg = agga_ref[0] + aggb_ref[0]
    hv1 = jnp.maximum(
        hv_ref[...] + jnp.dot(agg, wu[...], preferred_element_type=jnp.float32)
        + bu[...], 0.0)
    hv1_ref[...] = hv1
    hr_ref[...] = _pad_lanes(jnp.dot(hv1, wra[...], preferred_element_type=jnp.float32))
    hc_ref[...] = _pad_lanes(jnp.dot(hv1, wrb[...], preferred_element_type=jnp.float32))


def _agg_spec():
    return pl.BlockSpec((1, RB, HID), lambda i: (0, i, 0))


def _hv_update(hv, aggp, p):
    wm = p['gl_Wm1']
    wra, wrb = wm[:HID], wm[HID:2 * HID]
    args = (hv, aggp[0:1], aggp[1:2], p['gl_Wu0'], p['gl_bu0'].reshape(1, -1), wra, wrb)
    specs = [_rows(RB, HID), _agg_spec(), _agg_spec()] + [_full(a.shape) for a in args[3:]]
    return pl.pallas_call(
        _upd_body,
        grid=(N // RB,),
        in_specs=specs,
        out_specs=[_rows(RB, HID), _rows(RB, HIDP), _rows(RB, HIDP)],
        out_shape=[jax.ShapeDtypeStruct((N, HID), jnp.float32),
                   jax.ShapeDtypeStruct((N, HIDP), jnp.float32),
                   jax.ShapeDtypeStruct((N, HIDP), jnp.float32)],
    )(*args)


def _fin_body(hv_ref, agga_ref, aggb_ref, zl_ref, wu, bu, wout, bout, gamma,
              w0, b0, w1, b1, w2, b2, w3, b3, wg0,
              zt_ref, xhat_ref, u0_ref):
    agg = agga_ref[0] + aggb_ref[0]
    hv2 = jnp.maximum(
        hv_ref[...] + jnp.dot(agg, wu[...], preferred_element_type=jnp.float32)
        + bu[...], 0.0)
    zg = jnp.dot(hv2, wout[...], preferred_element_type=jnp.float32) + bout[...]
    zt = gamma[...] * zg + zl_ref[...]
    zt_ref[...] = zt
    u0_ref[...] = jnp.dot(zt, wg0[...], preferred_element_type=jnp.float32)
    h1 = jnp.maximum(jnp.dot(zt, w0[...], preferred_element_type=jnp.float32) + b0[...], 0.0)
    h2 = jnp.maximum(jnp.dot(h1, w1[...], preferred_element_type=jnp.float32) + b1[...], 0.0)
    h3 = jnp.maximum(jnp.dot(h2, w2[...], preferred_element_type=jnp.float32) + b2[...], 0.0)
    xhat_ref[...] = jnp.dot(h3, w3[...], preferred_element_type=jnp.float32) + b3[...]


def _final_update(hv1, aggp1, z_l, p):
    args = [hv1, aggp1[0:1], aggp1[1:2], z_l, p['gl_Wu1'], p['gl_bu1'].reshape(1, -1),
            p['gl_Wout'], p['gl_bout'].reshape(1, -1), p['gamma'].reshape(1, 1)]
    for i in range(4):
        args += [p['ae_dec_W%d' % i], p['ae_dec_b%d' % i].reshape(1, -1)]
    args.append(p['gae_dec_W0'])
    specs = [_rows(RB, HID), _agg_spec(), _agg_spec(), _rows(RB, N_Z)] \
        + [_full(a.shape) for a in args[4:]]
    return pl.pallas_call(
        _fin_body,
        grid=(N // RB,),
        in_specs=specs,
        out_specs=[_rows(RB, N_Z), _rows(RB, N_INPUT), _rows(RB, 256)],
        out_shape=[jax.ShapeDtypeStruct((N, N_Z), jnp.float32),
                   jax.ShapeDtypeStruct((N, N_INPUT), jnp.float32),
                   jax.ShapeDtypeStruct((N, 256), jnp.float32)],
    )(*args)


AB = 512


def _adj_body(zi_ref, zj_ref, hi_ref, hj_ref, o_ref):
    s1 = jax.lax.dot_general(zi_ref[...], zj_ref[...],
                             (((1,), (1,)), ((), ())),
                             preferred_element_type=jnp.float32)
    s2 = jax.lax.dot_general(hi_ref[...].astype(jnp.bfloat16),
                             hj_ref[...].astype(jnp.bfloat16),
                             (((1,), (1,)), ((), ())),
                             preferred_element_type=jnp.float32)
    o_ref[...] = jax.nn.sigmoid(s1) + jax.nn.sigmoid(s2)


def _adj_hat(z_egae, z_hat):
    return pl.pallas_call(
        _adj_body,
        grid=(N // AB, N // AB),
        in_specs=[pl.BlockSpec((AB, N_Z), lambda i, j: (i, 0)),
                  pl.BlockSpec((AB, N_Z), lambda i, j: (j, 0)),
                  pl.BlockSpec((AB, N_INPUT), lambda i, j: (i, 0)),
                  pl.BlockSpec((AB, N_INPUT), lambda i, j: (j, 0))],
        out_specs=pl.BlockSpec((AB, AB), lambda i, j: (i, j)),
        out_shape=jax.ShapeDtypeStruct((N, N), jnp.float32),
    )(z_egae, z_egae, z_hat, z_hat)


def _soft_body(z1_ref, z2_ref, z3_ref, cl_ref, cn_ref, q1_ref, q2_ref, q3_ref):
    cl = cl_ref[...]
    cn = cn_ref[...]
    for z_ref, q_ref in ((z1_ref, q1_ref), (z2_ref, q2_ref), (z3_ref, q3_ref)):
        z = z_ref[...]
        zn = jnp.sum(z * z, axis=1, keepdims=True)
        d = zn + cn - 2.0 * jax.lax.dot_general(
            z, cl, (((1,), (1,)), ((), ())), preferred_element_type=jnp.float32)
        q = 1.0 / (1.0 + d / V)
        q_ref[...] = q / jnp.sum(q, axis=1, keepdims=True)


def _softs(z_tilde, z_ae, z_egae, cl):
    cn = jnp.sum(cl * cl, axis=1).reshape(1, -1)
    nc = cl.shape[0]
    return pl.pallas_call(
        _soft_body,
        grid=(N // AB,),
        in_specs=[_rows(AB, N_Z)] * 3 + [_full(cl.shape), _full(cn.shape)],
        out_specs=[_rows(AB, nc)] * 3,
        out_shape=[jax.ShapeDtypeStruct((N, nc), jnp.float32)] * 3,
    )(z_tilde, z_ae, z_egae, cl, cn)


def kernel(x, adj_values, distance, params, edge_index):
    p = params
    rows = edge_index[0].astype(jnp.int32)
    cols = edge_index[1].astype(jnp.int32)

    a_flat, ef = _sc_build(rows, cols, adj_values, distance.reshape(-1))
    A = a_flat.reshape(N, N)

    z_ae, g0 = _ae_encode(x, p)

    g1 = _spmm_mm(A, g0, p['gae_enc_W1'], relu=True)
    g2 = _spmm_mm(A, g1, p['gae_enc_W2'], relu=True)
    z_egae = _spmm_mm(A, g2)

    z_i, hv, hr0, hc0 = _combine_hv(z_ae, z_egae, p)
    z_l = _spmm_mm(A, z_i)

    he0, he1 = _he_tables(ef, p)

    aggp0 = _edge_mp(rows, cols, hr0, hc0, he0)
    hv1, hr1, hc1 = _hv_update(hv, aggp0, p)

    aggp1 = _edge_mp(rows, cols, hr1, hc1, he1)

    z_tilde, x_hat, u0 = _final_update(hv1, aggp1, z_l, p)

    u1 = _spmm_mm(A, u0, p['gae_dec_W1'], relu=True)
    u2 = _spmm_mm(A, u1, p['gae_dec_W2'], relu=True)
    z_hat = _spmm_mm(A, u2)

    adj_hat = _adj_hat(z_egae, z_hat)
    q1, q2, q3 = _softs(z_tilde, z_ae, z_egae, p['cluster'])

    return (x_hat, z_hat, adj_hat, z_ae, z_egae, q1, q2, q3, z_tilde)


# speedup vs baseline: 5.7128x; 1.0048x over previous
import functools

import jax
import jax.numpy as jnp
from jax import lax
from jax.experimental import pallas as pl
from jax.experimental.pallas import tpu as pltpu
from jax.experimental.pallas import tpu_sc as plsc

N = 4096
E = 65536
N_INPUT = 512
N_Z = 20
HID = 64
V = 1.0

RB = 256
EB = 1024


def _full(shape):
    return pl.BlockSpec(shape, lambda *_: (0,) * len(shape))


def _rows(bs, ncols):
    return pl.BlockSpec((bs, ncols), lambda i: (i, 0))


GR = 64
NG = N // GR
ACC_LEN = (GR + 2) * N
DUMP = GR * N
FRT = GR // 16
EPT = E // 16
EFT = E // 32
ZB = 16384
TPW = ACC_LEN // 16


NCH = EPT // 128


def _abuild_body(rows_hbm, cols_hbm, vals_hbm, dist_hbm, a_hbm, ef_hbm,
                 rows_v, cols_v, vals_v, lin_v, idx_v, ef_v, zbuf, acc,
                 ssem, fsem):
    c = lax.axis_index("c")
    s = lax.axis_index("s")
    ebase = s * EPT
    pltpu.sync_copy(rows_hbm.at[pl.ds(ebase, EPT)], rows_v)
    pltpu.sync_copy(cols_hbm.at[pl.ds(ebase, EPT)], cols_v)
    pltpu.sync_copy(vals_hbm.at[pl.ds(ebase, EPT)], vals_v)

    def zb_init(i, _):
        zbuf[pl.ds(i * 16, 16)] = jnp.zeros((16,), jnp.float32)
        return 0
    lax.fori_loop(0, ZB // 16, zb_init, 0)

    def lin_init(i, _):
        o = i * 16
        lin_v[pl.ds(o, 16)] = rows_v[pl.ds(o, 16)] * N + cols_v[pl.ds(o, 16)]
        return 0
    lax.fori_loop(0, EPT // 16, lin_init, 0)

    off0 = c * EFT

    def ef_chunk(j, _):
        def lanes(l, _):
            o = j * 128 + l * 16
            row = idx_v.at[j]
            row[pl.ds(l * 16, 16)] = lin_v[pl.ds(off0 + o, 16)]
            return 0
        lax.fori_loop(0, 8, lanes, 0)
        pltpu.sync_copy(dist_hbm.at[idx_v.at[j]], ef_v)
        pltpu.sync_copy(ef_v, ef_hbm.at[pl.ds(ebase + off0 + j * 128, 128)])
        return 0
    lax.fori_loop(0, EFT // 128, ef_chunk, 0)

    def group(g, _):
        lo = (g * 2 + c) * GR
        base = s * TPW
        for t in range(TPW // ZB):
            pltpu.sync_copy(zbuf, acc.at[pl.ds(base + t * ZB, ZB)])
        if TPW % ZB:
            pltpu.sync_copy(zbuf.at[pl.ds(0, TPW % ZB)],
                            acc.at[pl.ds(base + (TPW // ZB) * ZB, TPW % ZB)])
        plsc.subcore_barrier()

        def chunk_fill(j):
            def lanes(l, _):
                o = j * 128 + l * 16
                gl = lin_v[pl.ds(o, 16)]
                rel = gl - lo * N
                ing = (rel >= 0) & (rel < GR * N)
                row = idx_v.at[j]
                row[pl.ds(l * 16, 16)] = jnp.where(
                    ing, rel, DUMP + (gl & (N - 1)))
                return 0
            lax.fori_loop(0, 8, lanes, 0)
            return pltpu.async_copy(vals_v.at[pl.ds(j * 128, 128)],
                                    acc.at[idx_v.at[j]], ssem, add=True)

        for jb in range(0, NCH, 8):
            pend = [chunk_fill(j) for j in range(jb, jb + 8)]
            for d in pend:
                d.wait()
        plsc.subcore_barrier()

        pend = [pltpu.async_copy(acc.at[pl.ds((s * FRT + r) * N, N)],
                                 a_hbm.at[lo + s * FRT + r], fsem)
                for r in range(FRT)]
        for d in pend:
            d.wait()
        plsc.subcore_barrier()
        return 0
    lax.fori_loop(0, NG // 2, group, 0)


def _sc_build(rows, cols, vals, dist_flat):
    mesh = plsc.VectorSubcoreMesh(core_axis_name="c", subcore_axis_name="s",
                                  num_cores=2, num_subcores=16)
    f = pl.kernel(
        _abuild_body,
        out_type=[jax.ShapeDtypeStruct((N, N), jnp.float32),
                  jax.ShapeDtypeStruct((E,), jnp.float32)],
        mesh=mesh,
        scratch_types=[
            pltpu.VMEM((EPT,), jnp.int32),
            pltpu.VMEM((EPT,), jnp.int32),
            pltpu.VMEM((EPT,), jnp.float32),
            pltpu.VMEM((EPT,), jnp.int32),
            pltpu.VMEM((NCH, 128), jnp.int32),
            pltpu.VMEM((128,), jnp.float32),
            pltpu.VMEM((ZB,), jnp.float32),
            pltpu.VMEM_SHARED((ACC_LEN,), jnp.float32),
            pltpu.SemaphoreType.DMA,
            pltpu.SemaphoreType.DMA,
        ],
    )
    return f(rows, cols, vals, dist_flat)


EMT = E // 32
ECH = EMT // 128


HIDP = 128


def _edge_mp_body(rows_hbm, cols_hbm, hr_hbm, hc_hbm, he_hbm, agg_hbm,
                  ridx, cidx, ga0, ga1, gb0, gb1, mb0, mb1, m640, m641,
                  zbuf, acc, gsem0, gsem1, ssem0, ssem1):
    ga = (ga0, ga1)
    gb = (gb0, gb1)
    mb = (mb0, mb1)
    m64 = (m640, m641)
    gsem = (gsem0, gsem1)
    ssem = (ssem0, ssem1)
    c = lax.axis_index("c")
    s = lax.axis_index("s")
    tid = s * 2 + c
    ebase = tid * EMT

    def ld(j, _):
        pltpu.sync_copy(rows_hbm.at[pl.ds(ebase + j * 128, 128)], ridx.at[j])
        pltpu.sync_copy(cols_hbm.at[pl.ds(ebase + j * 128, 128)], cidx.at[j])
        return 0
    lax.fori_loop(0, ECH, ld, 0)

    def zb_init(i, _):
        def inner(k, _):
            zbuf[i, pl.ds(k * 16, 16)] = jnp.zeros((16,), jnp.float32)
            return 0
        lax.fori_loop(0, HID // 16, inner, 0)
        return 0
    lax.fori_loop(0, 128, zb_init, 0)
    pltpu.sync_copy(zbuf, acc.at[pl.ds(s * 256, 128)])
    pltpu.sync_copy(zbuf, acc.at[pl.ds(s * 256 + 128, 128)])
    plsc.subcore_barrier()

    def chunk(j, _):
        pltpu.sync_copy(hr_hbm.at[ridx.at[j]], ga[0])
        pltpu.sync_copy(hc_hbm.at[cidx.at[j]], gb[0])
        pltpu.sync_copy(he_hbm.at[pl.ds(ebase + j * 128, 128)], mb[0])

        def row(i, _):
            for k in range(HID // 16):
                d = pl.ds(k * 16, 16)
                m640[i, d] = jnp.maximum(ga0[i, d] + gb0[i, d] + mb0[i, d], 0.0)
            return 0
        lax.fori_loop(0, 128, row, 0)
        pltpu.sync_copy(m640, acc.at[cidx.at[j]], add=True)
        return 0
    lax.fori_loop(0, ECH, chunk, 0)
    plsc.subcore_barrier()
    pltpu.sync_copy(acc.at[pl.ds(s * 256, 256)],
                    agg_hbm.at[c, pl.ds(s * 256, 256)])


def _edge_mp(rows, cols, hr, hc, he):
    mesh = plsc.VectorSubcoreMesh(core_axis_name="c", subcore_axis_name="s",
                                  num_cores=2, num_subcores=16)
    f = pl.kernel(
        _edge_mp_body,
        out_type=jax.ShapeDtypeStruct((2, N, HID), jnp.float32),
        mesh=mesh,
        scratch_types=[
            pltpu.VMEM((ECH, 128), jnp.int32),
            pltpu.VMEM((ECH, 128), jnp.int32),
            pltpu.VMEM((128, HIDP), jnp.float32),
            pltpu.VMEM((128, HIDP), jnp.float32),
            pltpu.VMEM((128, HIDP), jnp.float32),
            pltpu.VMEM((128, HIDP), jnp.float32),
            pltpu.VMEM((128, HIDP), jnp.float32),
            pltpu.VMEM((128, HIDP), jnp.float32),
            pltpu.VMEM((128, HID), jnp.float32),
            pltpu.VMEM((128, HID), jnp.float32),
            pltpu.VMEM((128, HID), jnp.float32),
            pltpu.VMEM_SHARED((N, HID), jnp.float32),
            pltpu.SemaphoreType.DMA,
            pltpu.SemaphoreType.DMA,
            pltpu.SemaphoreType.DMA,
            pltpu.SemaphoreType.DMA,
        ],
    )
    return f(rows, cols, hr, hc, he)


def _cast_body(a_ref, o_ref):
    o_ref[...] = a_ref[...].astype(jnp.bfloat16)


def _cast_bf16(a):
    return pl.pallas_call(
        _cast_body,
        grid=(N // RB,),
        in_specs=[_rows(RB, N)],
        out_specs=_rows(RB, N),
        out_shape=jax.ShapeDtypeStruct((N, N), jnp.bfloat16),
    )(a)


def _spmm_body(a_ref, h_ref, w_ref, o_ref, *, relu):
    s = jnp.dot(a_ref[...], h_ref[...].astype(jnp.bfloat16),
                preferred_element_type=jnp.float32)
    if relu:
        s = jnp.maximum(s, 0.0)
    if w_ref is not None:
        s = jnp.dot(s, w_ref[...], preferred_element_type=jnp.float32)
    o_ref[...] = s


def _spmm_mm(a, h, w=None, relu=False):
    d = h.shape[1]
    dout = d if w is None else w.shape[1]
    if w is None:
        body = functools.partial(lambda ar, hr, orr, relu: _spmm_body(ar, hr, None, orr, relu=relu), relu=relu)
        in_specs = [_rows(RB, N), _full((N, d))]
        args = (a, h)
    else:
        body = functools.partial(_spmm_body, relu=relu)
        in_specs = [_rows(RB, N), _full((N, d)), _full(w.shape)]
        args = (a, h, w)
    return pl.pallas_call(
        body,
        grid=(N // RB,),
        in_specs=in_specs,
        out_specs=_rows(RB, dout),
        out_shape=jax.ShapeDtypeStruct((N, dout), jnp.float32),
    )(*args)


def _ae_enc_body(x_ref, w0, b0, w1, b1, w2, b2, w3, b3, wg0, zae_ref, g0_ref):
    h = x_ref[...]
    g0_ref[...] = jnp.dot(h, wg0[...], preferred_element_type=jnp.float32)
    h1 = jnp.maximum(jnp.dot(h, w0[...], preferred_element_type=jnp.float32) + b0[...], 0.0)
    h2 = jnp.maximum(jnp.dot(h1, w1[...], preferred_element_type=jnp.float32) + b1[...], 0.0)
    h3 = jnp.maximum(jnp.dot(h2, w2[...], preferred_element_type=jnp.float32) + b2[...], 0.0)
    zae_ref[...] = jnp.dot(h3, w3[...], preferred_element_type=jnp.float32) + b3[...]


def _ae_encode(x, p):
    ws = []
    specs = [_rows(RB, N_INPUT)]
    for i in range(4):
        w = p['ae_enc_W%d' % i]
        b = p['ae_enc_b%d' % i].reshape(1, -1)
        ws += [w, b]
        specs += [_full(w.shape), _full(b.shape)]
    ws.append(p['gae_enc_W0'])
    specs.append(_full(p['gae_enc_W0'].shape))
    return pl.pallas_call(
        _ae_enc_body,
        grid=(N // RB,),
        in_specs=specs,
        out_specs=[_rows(RB, N_Z), _rows(RB, 128)],
        out_shape=[jax.ShapeDtypeStruct((N, N_Z), jnp.float32),
                   jax.ShapeDtypeStruct((N, 128), jnp.float32)],
    )(x, *ws)


def _pad_lanes(v):
    return jnp.concatenate([v, jnp.zeros_like(v)], axis=1)


def _comb_body(zae_ref, zegae_ref, a_ref, wv, bv, wra, wrb,
               zi_ref, hv_ref, hr_ref, hc_ref):
    a = a_ref[...]
    zi = a * zae_ref[...] + (1.0 - a) * zegae_ref[...]
    zi_ref[...] = zi
    hv = jnp.maximum(jnp.dot(zi, wv[...], preferred_element_type=jnp.float32) + bv[...], 0.0)
    hv_ref[...] = hv
    hr_ref[...] = _pad_lanes(jnp.dot(hv, wra[...], preferred_element_type=jnp.float32))
    hc_ref[...] = _pad_lanes(jnp.dot(hv, wrb[...], preferred_element_type=jnp.float32))


def _combine_hv(z_ae, z_egae, p):
    wm = p['gl_Wm0']
    wra, wrb = wm[:HID], wm[HID:2 * HID]
    args = (z_ae, z_egae, p['a'], p['gl_Wv'], p['gl_bv'].reshape(1, -1), wra, wrb)
    specs = [_rows(RB, N_Z), _rows(RB, N_Z), _rows(RB, N_Z),
             _full(args[3].shape), _full(args[4].shape), _full(wra.shape), _full(wrb.shape)]
    return pl.pallas_call(
        _comb_body,
        grid=(N // RB,),
        in_specs=specs,
        out_specs=[_rows(RB, N_Z), _rows(RB, HID), _rows(RB, HIDP), _rows(RB, HIDP)],
        out_shape=[jax.ShapeDtypeStruct((N, N_Z), jnp.float32),
                   jax.ShapeDtypeStruct((N, HID), jnp.float32),
                   jax.ShapeDtypeStruct((N, HIDP), jnp.float32),
                   jax.ShapeDtypeStruct((N, HIDP), jnp.float32)],
    )(*args)


def _he_body(ef_ref, we, be, wc0, bm0, wc1, bm1, he0_ref, he1_ref):
    he = jnp.maximum(ef_ref[...] * we[...] + be[...], 0.0)
    he0_ref[...] = _pad_lanes(jnp.dot(he, wc0[...], preferred_element_type=jnp.float32) + bm0[...])
    he1_ref[...] = _pad_lanes(jnp.dot(he, wc1[...], preferred_element_type=jnp.float32) + bm1[...])


def _he_tables(ef, p):
    args = (ef.reshape(E, 1), p['gl_We'], p['gl_be'].reshape(1, -1),
            p['gl_Wm0'][2 * HID:], p['gl_bm0'].reshape(1, -1),
            p['gl_Wm1'][2 * HID:], p['gl_bm1'].reshape(1, -1))
    specs = [_rows(EB, 1)] + [_full(a.shape) for a in args[1:]]
    return pl.pallas_call(
        _he_body,
        grid=(E // EB,),
        in_specs=specs,
        out_specs=[_rows(EB, HIDP)] * 2,
        out_shape=[jax.ShapeDtypeStruct((E, HIDP), jnp.float32)] * 2,
    )(*args)


def _upd_body(hv_ref, agga_ref, aggb_ref, wu, bu, wra, wrb, hv1_ref, hr_ref, hc_ref):
    agg = agga_ref[0] + aggb_ref[0]
    hv1 = jnp.maximum(
        hv_ref[...] + jnp.dot(agg, wu[...], preferred_element_type=jnp.float32)
        + bu[...], 0.0)
    hv1_ref[...] = hv1
    hr_ref[...] = _pad_lanes(jnp.dot(hv1, wra[...], preferred_element_type=jnp.float32))
    hc_ref[...] = _pad_lanes(jnp.dot(hv1, wrb[...], preferred_element_type=jnp.float32))


def _agg_spec():
    return pl.BlockSpec((1, RB, HID), lambda i: (0, i, 0))


def _hv_update(hv, aggp, p):
    wm = p['gl_Wm1']
    wra, wrb = wm[:HID], wm[HID:2 * HID]
    args = (hv, aggp[0:1], aggp[1:2], p['gl_Wu0'], p['gl_bu0'].reshape(1, -1), wra, wrb)
    specs = [_rows(RB, HID), _agg_spec(), _agg_spec()] + [_full(a.shape) for a in args[3:]]
    return pl.pallas_call(
        _upd_body,
        grid=(N // RB,),
        in_specs=specs,
        out_specs=[_rows(RB, HID), _rows(RB, HIDP), _rows(RB, HIDP)],
        out_shape=[jax.ShapeDtypeStruct((N, HID), jnp.float32),
                   jax.ShapeDtypeStruct((N, HIDP), jnp.float32),
                   jax.ShapeDtypeStruct((N, HIDP), jnp.float32)],
    )(*args)


def _fin_body(hv_ref, agga_ref, aggb_ref, zl_ref, wu, bu, wout, bout, gamma,
              w0, b0, w1, b1, w2, b2, w3, b3, wg0,
              zt_ref, xhat_ref, u0_ref):
    agg = agga_ref[0] + aggb_ref[0]
    hv2 = jnp.maximum(
        hv_ref[...] + jnp.dot(agg, wu[...], preferred_element_type=jnp.float32)
        + bu[...], 0.0)
    zg = jnp.dot(hv2, wout[...], preferred_element_type=jnp.float32) + bout[...]
    zt = gamma[...] * zg + zl_ref[...]
    zt_ref[...] = zt
    u0_ref[...] = jnp.dot(zt, wg0[...], preferred_element_type=jnp.float32)
    h1 = jnp.maximum(jnp.dot(zt, w0[...], preferred_element_type=jnp.float32) + b0[...], 0.0)
    h2 = jnp.maximum(jnp.dot(h1, w1[...], preferred_element_type=jnp.float32) + b1[...], 0.0)
    h3 = jnp.maximum(jnp.dot(h2, w2[...], preferred_element_type=jnp.float32) + b2[...], 0.0)
    xhat_ref[...] = jnp.dot(h3, w3[...], preferred_element_type=jnp.float32) + b3[...]


def _final_update(hv1, aggp1, z_l, p):
    args = [hv1, aggp1[0:1], aggp1[1:2], z_l, p['gl_Wu1'], p['gl_bu1'].reshape(1, -1),
            p['gl_Wout'], p['gl_bout'].reshape(1, -1), p['gamma'].reshape(1, 1)]
    for i in range(4):
        args += [p['ae_dec_W%d' % i], p['ae_dec_b%d' % i].reshape(1, -1)]
    args.append(p['gae_dec_W0'])
    specs = [_rows(RB, HID), _agg_spec(), _agg_spec(), _rows(RB, N_Z)] \
        + [_full(a.shape) for a in args[4:]]
    return pl.pallas_call(
        _fin_body,
        grid=(N // RB,),
        in_specs=specs,
        out_specs=[_rows(RB, N_Z), _rows(RB, N_INPUT), _rows(RB, 256)],
        out_shape=[jax.ShapeDtypeStruct((N, N_Z), jnp.float32),
                   jax.ShapeDtypeStruct((N, N_INPUT), jnp.float32),
                   jax.ShapeDtypeStruct((N, 256), jnp.float32)],
    )(*args)


AB = 512


def _adj_body(zi_ref, zj_ref, hi_ref, hj_ref, o_ref):
    s1 = jax.lax.dot_general(zi_ref[...], zj_ref[...],
                             (((1,), (1,)), ((), ())),
                             preferred_element_type=jnp.float32)
    s2 = jax.lax.dot_general(hi_ref[...].astype(jnp.bfloat16),
                             hj_ref[...].astype(jnp.bfloat16),
                             (((1,), (1,)), ((), ())),
                             preferred_element_type=jnp.float32)
    o_ref[...] = jax.nn.sigmoid(s1) + jax.nn.sigmoid(s2)


def _adj_hat(z_egae, z_hat):
    return pl.pallas_call(
        _adj_body,
        grid=(N // AB, N // AB),
        in_specs=[pl.BlockSpec((AB, N_Z), lambda i, j: (i, 0)),
                  pl.BlockSpec((AB, N_Z), lambda i, j: (j, 0)),
                  pl.BlockSpec((AB, N_INPUT), lambda i, j: (i, 0)),
                  pl.BlockSpec((AB, N_INPUT), lambda i, j: (j, 0))],
        out_specs=pl.BlockSpec((AB, AB), lambda i, j: (i, j)),
        out_shape=jax.ShapeDtypeStruct((N, N), jnp.float32),
    )(z_egae, z_egae, z_hat, z_hat)


def _soft_body(z1_ref, z2_ref, z3_ref, cl_ref, cn_ref, q1_ref, q2_ref, q3_ref):
    cl = cl_ref[...]
    cn = cn_ref[...]
    for z_ref, q_ref in ((z1_ref, q1_ref), (z2_ref, q2_ref), (z3_ref, q3_ref)):
        z = z_ref[...]
        zn = jnp.sum(z * z, axis=1, keepdims=True)
        d = zn + cn - 2.0 * jax.lax.dot_general(
            z, cl, (((1,), (1,)), ((), ())), preferred_element_type=jnp.float32)
        q = 1.0 / (1.0 + d / V)
        q_ref[...] = q / jnp.sum(q, axis=1, keepdims=True)


def _softs(z_tilde, z_ae, z_egae, cl):
    cn = jnp.sum(cl * cl, axis=1).reshape(1, -1)
    nc = cl.shape[0]
    return pl.pallas_call(
        _soft_body,
        grid=(N // AB,),
        in_specs=[_rows(AB, N_Z)] * 3 + [_full(cl.shape), _full(cn.shape)],
        out_specs=[_rows(AB, nc)] * 3,
        out_shape=[jax.ShapeDtypeStruct((N, nc), jnp.float32)] * 3,
    )(z_tilde, z_ae, z_egae, cl, cn)


def kernel(x, adj_values, distance, params, edge_index):
    p = params
    rows = edge_index[0].astype(jnp.int32)
    cols = edge_index[1].astype(jnp.int32)

    a_f32, ef = _sc_build(rows, cols, adj_values, distance.reshape(-1))
    A = _cast_bf16(a_f32)

    z_ae, g0 = _ae_encode(x, p)

    g1 = _spmm_mm(A, g0, p['gae_enc_W1'], relu=True)
    g2 = _spmm_mm(A, g1, p['gae_enc_W2'], relu=True)
    z_egae = _spmm_mm(A, g2)

    z_i, hv, hr0, hc0 = _combine_hv(z_ae, z_egae, p)
    z_l = _spmm_mm(A, z_i)

    he0, he1 = _he_tables(ef, p)

    aggp0 = _edge_mp(rows, cols, hr0, hc0, he0)
    hv1, hr1, hc1 = _hv_update(hv, aggp0, p)

    aggp1 = _edge_mp(rows, cols, hr1, hc1, he1)

    z_tilde, x_hat, u0 = _final_update(hv1, aggp1, z_l, p)

    u1 = _spmm_mm(A, u0, p['gae_dec_W1'], relu=True)
    u2 = _spmm_mm(A, u1, p['gae_dec_W2'], relu=True)
    z_hat = _spmm_mm(A, u2)

    adj_hat = _adj_hat(z_egae, z_hat)
    q1, q2, q3 = _softs(z_tilde, z_ae, z_egae, p['cluster'])

    return (x_hat, z_hat, adj_hat, z_ae, z_egae, q1, q2, q3, z_tilde)
